# Initial kernel scaffold; baseline (speedup 1.0000x reference)
import jax
import jax.numpy as jnp
from jax.experimental import pallas as pl


def kernel(all_emb0, all_emb1, all_emb2, Wf0, Wf1, Wf2, W_ext1, b_ext1, W_ext2, b_ext2, W_enc1, W_enc2, W_out, node_ids, edge_src, edge_dst, e_hop, labels):
    raise NotImplementedError("write your pallas kernel here")


# speedup vs baseline: 6.6801x; 6.6801x over previous
import functools

import jax
import jax.numpy as jnp
from jax import lax
from jax.experimental import pallas as pl
from jax.experimental.pallas import tpu as pltpu
from jax.experimental.pallas import tpu_sc as plsc

_N_TOTAL = 50000
_N = 10000
_E = 160000
_B = 16
_NPG = 625
_IN = 128
_HID = 32
_ENC = 64
_NC = 2
_NS = 16
_NW = _NC * _NS
_EPW = _E // _NW
_CHUNK = 128
_NFULL = _EPW // _CHUNK
_TAIL = _EPW - _NFULL * _CHUNK
_RPT = _N // _NS
_GPW = 312
_GCH = 104


def _sc_mesh():
    return plsc.VectorSubcoreMesh(
        core_axis_name="c", subcore_axis_name="s",
        num_cores=_NC, num_subcores=_NS)


_SC_PARAMS = pltpu.CompilerParams(use_tc_tiling_on_sc=False,
                                  needs_layout_passes=False)


def _gather3(emb0, emb1, emb2, nid):
    @functools.partial(
        pl.kernel,
        out_type=(
            jax.ShapeDtypeStruct((_N, 128), jnp.float32),
            jax.ShapeDtypeStruct((_N, 256), jnp.float32),
            jax.ShapeDtypeStruct((_N, 64), jnp.float32),
        ),
        mesh=_sc_mesh(),
        scratch_types=[
            pltpu.VMEM((_GCH,), jnp.int32),
            pltpu.VMEM((16,), jnp.int32),
            pltpu.VMEM((_GCH, 128), jnp.float32),
            pltpu.VMEM((_GCH, 256), jnp.float32),
            pltpu.VMEM((_GCH, 64), jnp.float32),
            pltpu.VMEM((16, 128), jnp.float32),
            pltpu.VMEM((16, 256), jnp.float32),
            pltpu.VMEM((16, 64), jnp.float32),
            pltpu.SemaphoreType.DMA,
        ],
        compiler_params=_SC_PARAMS,
    )
    def k(e0, e1, e2, nid_h, g0, g1, g2,
          idx_v, idx_t, b0, b1, b2, t0, t1, t2, sem):
        wid = lax.axis_index("s") * _NC + lax.axis_index("c")
        base = wid * _GPW
        for ci in range(_GPW // _GCH):
            off = base + ci * _GCH
            pltpu.sync_copy(nid_h.at[pl.ds(off, _GCH)], idx_v)
            c0 = pltpu.async_copy(e0.at[idx_v], b0, sem)
            c1 = pltpu.async_copy(e1.at[idx_v], b1, sem)
            c2 = pltpu.async_copy(e2.at[idx_v], b2, sem)
            c0.wait()
            c1.wait()
            c2.wait()
            pltpu.sync_copy(b0, g0.at[pl.ds(off, _GCH)])
            pltpu.sync_copy(b1, g1.at[pl.ds(off, _GCH)])
            pltpu.sync_copy(b2, g2.at[pl.ds(off, _GCH)])

        @pl.when(wid == 0)
        def _tail():
            off = _NW * _GPW
            pltpu.sync_copy(nid_h.at[pl.ds(off, 16)], idx_t)
            c0 = pltpu.async_copy(e0.at[idx_t], t0, sem)
            c1 = pltpu.async_copy(e1.at[idx_t], t1, sem)
            c2 = pltpu.async_copy(e2.at[idx_t], t2, sem)
            c0.wait()
            c1.wait()
            c2.wait()
            pltpu.sync_copy(t0, g0.at[pl.ds(off, 16)])
            pltpu.sync_copy(t1, g1.at[pl.ds(off, 16)])
            pltpu.sync_copy(t2, g2.at[pl.ds(off, 16)])

    return k(emb0, emb1, emb2, nid)


def _dense_node(G0, G1, G2, Wf0, Wf1, Wf2, W1a, W1b, Wenc1):
    blk = 1000

    def body(g0, g1, g2, wf0, wf1, wf2, w1a, w1b, we1,
             p0, q0, p1, q1, xw, f2o):
        f0 = jnp.maximum(jnp.dot(g0[...], wf0[...],
                                 preferred_element_type=jnp.float32), 0.0)
        f1 = jnp.maximum(jnp.dot(g1[...], wf1[...],
                                 preferred_element_type=jnp.float32), 0.0)
        f2 = jnp.maximum(jnp.dot(g2[...], wf2[...],
                                 preferred_element_type=jnp.float32), 0.0)
        p0[...] = jnp.dot(f0, w1a[...], preferred_element_type=jnp.float32)
        q0[...] = jnp.dot(f1, w1b[...], preferred_element_type=jnp.float32)
        p1[...] = jnp.dot(f1, w1a[...], preferred_element_type=jnp.float32)
        q1[...] = jnp.dot(f2, w1b[...], preferred_element_type=jnp.float32)
        xw[...] = jnp.dot(g0[...], we1[...],
                          preferred_element_type=jnp.float32)
        f2o[...] = f2

    full = lambda a, b: pl.BlockSpec((a, b), lambda i: (0, 0))
    row = lambda w: pl.BlockSpec((blk, w), lambda i: (i, 0))
    outs = [jax.ShapeDtypeStruct((_N, _HID), jnp.float32)] * 4 + [
        jax.ShapeDtypeStruct((_N, _ENC), jnp.float32),
        jax.ShapeDtypeStruct((_N, _HID), jnp.float32)]
    return pl.pallas_call(
        body,
        grid=(_N // blk,),
        in_specs=[row(128), row(256), row(64),
                  full(128, _HID), full(256, _HID), full(64, _HID),
                  full(_HID, _HID), full(_HID, _HID), full(128, _ENC)],
        out_specs=[row(_HID), row(_HID), row(_HID), row(_HID),
                   row(_ENC), row(_HID)],
        out_shape=outs,
    )(G0, G1, G2, Wf0, Wf1, Wf2, W1a, W1b, Wenc1)


def _rb_tc(F2o, W1c, b1r):
    def body(f, w, b, o):
        o[...] = jnp.dot(f[...], w[...],
                         preferred_element_type=jnp.float32) + b[...]

    return pl.pallas_call(
        body,
        out_shape=jax.ShapeDtypeStruct((_B, _HID), jnp.float32),
    )(F2o, W1c, b1r)


def _dense_mid(HP, W):
    blk = 1000

    def body(hp, w, o):
        h = jnp.maximum(hp[0] + hp[1], 0.0)
        o[...] = jnp.dot(h, w[...], preferred_element_type=jnp.float32)

    return pl.pallas_call(
        body,
        grid=(_N // blk,),
        in_specs=[pl.BlockSpec((2, blk, _ENC), lambda i: (0, i, 0)),
                  pl.BlockSpec((_ENC, _ENC), lambda i: (0, 0))],
        out_specs=pl.BlockSpec((blk, _ENC), lambda i: (i, 0)),
        out_shape=jax.ShapeDtypeStruct((_N, _ENC), jnp.float32),
    )(HP, W)


def _final_tc(H2o, Wout):
    def body(h, w, o):
        o[...] = jnp.dot(h[0] + h[1], w[...],
                         preferred_element_type=jnp.float32)

    return pl.pallas_call(
        body,
        out_shape=jax.ShapeDtypeStruct((_B, 8), jnp.float32),
    )(H2o, Wout)


def _edge_pass(P, Q, V, RB1f, consts, esrc, edst, ehop, zrows, hop_sel):
    @functools.partial(
        pl.kernel,
        out_type=jax.ShapeDtypeStruct((_NC * _N, _ENC), jnp.float32),
        mesh=_sc_mesh(),
        scratch_types=[
            pltpu.VMEM((_CHUNK,), jnp.int32),
            pltpu.VMEM((_CHUNK,), jnp.int32),
            pltpu.VMEM((_CHUNK,), jnp.int32),
            pltpu.VMEM((16,), jnp.int32),
            pltpu.VMEM((16,), jnp.int32),
            pltpu.VMEM((16,), jnp.int32),
            pltpu.VMEM((_CHUNK, _HID), jnp.float32),
            pltpu.VMEM((_CHUNK, _HID), jnp.float32),
            pltpu.VMEM((_CHUNK, _ENC), jnp.float32),
            pltpu.VMEM((_CHUNK, _ENC), jnp.float32),
            pltpu.VMEM((16, _HID), jnp.float32),
            pltpu.VMEM((16, _HID), jnp.float32),
            pltpu.VMEM((16, _ENC), jnp.float32),
            pltpu.VMEM((16, _ENC), jnp.float32),
            pltpu.VMEM((512,), jnp.float32),
            pltpu.VMEM((512,), jnp.float32),
            pltpu.VMEM((48,), jnp.float32),
            pltpu.VMEM_SHARED((_N, _ENC), jnp.float32),
            pltpu.SemaphoreType.DMA,
        ],
        compiler_params=_SC_PARAMS,
    )
    def k(P_h, Q_h, V_h, rb_h, c_h, es_h, ed_h, eh_h, z_h, out_h,
          sidx, didx, hbuf, sidx16, didx16, hidx16,
          pbuf, qbuf, vbuf, mbuf, pbuf16, qbuf16, vbuf16, mbuf16,
          tt, rbv, cv, hacc, sem):
        cid = lax.axis_index("c")
        sid = lax.axis_index("s")
        wid = sid * _NC + cid
        pltpu.sync_copy(z_h.at[pl.ds(sid * _RPT, _RPT)],
                        hacc.at[pl.ds(sid * _RPT, _RPT)])
        pltpu.sync_copy(rb_h, rbv)
        pltpu.sync_copy(c_h, cv)
        plsc.subcore_barrier()

        lane = lax.iota(jnp.int32, 16)
        b = wid // 2
        rb_lo = rbv[pl.ds(pl.multiple_of(b * 32, 16), 16)]
        rb_hi = rbv[pl.ds(pl.multiple_of(b * 32 + 16, 16), 16)]
        w2_lo = cv[pl.ds(0, 16)]
        w2_hi = cv[pl.ds(16, 16)]
        b2s = cv[pl.ds(32, 16)][0]
        ebase = wid * _EPW

        def group(pb, qb, vb, mb, hv, e0):
            for le in range(16):
                e = e0 + le
                p_lo = pb[e, pl.ds(0, 16)]
                p_hi = pb[e, pl.ds(16, 16)]
                q_lo = qb[e, pl.ds(0, 16)]
                q_hi = qb[e, pl.ds(16, 16)]
                t_lo = jnp.maximum(p_lo + q_lo + rb_lo, 0.0)
                t_hi = jnp.maximum(p_hi + q_hi + rb_hi, 0.0)
                plsc.store_scatter(tt, [lane * 16 + le], t_lo)
                plsc.store_scatter(tt, [lane * 16 + (256 + le)], t_hi)
            att = jnp.full((16,), 0.0, jnp.float32) + b2s
            for kk in range(16):
                att = att + tt[pl.ds(kk * 16, 16)] * w2_lo[kk]
            for kk in range(16):
                att = att + tt[pl.ds((kk + 16) * 16, 16)] * w2_hi[kk]
            sig = 1.0 / (1.0 + jnp.exp(-att))
            gate = jnp.where(hv == hop_sel, sig, 0.0)
            for le in range(16):
                e = e0 + le
                g_s = gate[le]
                for j in range(_ENC // 16):
                    mb[e, pl.ds(j * 16, 16)] = vb[e, pl.ds(j * 16, 16)] * g_s

        def chunk_body(ci, _):
            base = pl.multiple_of(ebase + ci * _CHUNK, 8)
            pltpu.sync_copy(es_h.at[pl.ds(base, _CHUNK)], sidx)
            pltpu.sync_copy(ed_h.at[pl.ds(base, _CHUNK)], didx)
            pltpu.sync_copy(eh_h.at[pl.ds(base, _CHUNK)], hbuf)
            cp = pltpu.async_copy(P_h.at[sidx], pbuf, sem)
            cq = pltpu.async_copy(Q_h.at[didx], qbuf, sem)
            cvv = pltpu.async_copy(V_h.at[sidx], vbuf, sem)
            cp.wait()
            cq.wait()
            cvv.wait()

            def group_body(g, __):
                e0 = g * 16
                hv = hbuf[pl.ds(e0, 16)]
                group(pbuf, qbuf, vbuf, mbuf, hv, e0)
                return __

            lax.fori_loop(0, _CHUNK // 16, group_body, 0)
            pltpu.sync_copy(mbuf, hacc.at[didx], add=True)
            return _

        lax.fori_loop(0, _NFULL, chunk_body, 0)

        tbase = ebase + _NFULL * _CHUNK
        pltpu.sync_copy(es_h.at[pl.ds(tbase, 8)], sidx.at[pl.ds(0, 8)])
        pltpu.sync_copy(ed_h.at[pl.ds(tbase, 8)], didx.at[pl.ds(0, 8)])
        pltpu.sync_copy(eh_h.at[pl.ds(tbase, 8)], hbuf.at[pl.ds(0, 8)])
        pad = lane < 8
        sidx16[...] = jnp.where(pad, sidx[pl.ds(0, 16)], 0)
        didx16[...] = jnp.where(pad, didx[pl.ds(0, 16)], 0)
        hidx16[...] = jnp.where(pad, hbuf[pl.ds(0, 16)], -1)
        cp = pltpu.async_copy(P_h.at[sidx16], pbuf16, sem)
        cq = pltpu.async_copy(Q_h.at[didx16], qbuf16, sem)
        cvv = pltpu.async_copy(V_h.at[sidx16], vbuf16, sem)
        cp.wait()
        cq.wait()
        cvv.wait()
        group(pbuf16, qbuf16, vbuf16, mbuf16, hidx16[...], 0)
        pltpu.sync_copy(mbuf16, hacc.at[didx16], add=True)

        plsc.subcore_barrier()
        out_off = pl.multiple_of(cid * _N + sid * _RPT, 8)
        pltpu.sync_copy(hacc.at[pl.ds(sid * _RPT, _RPT)],
                        out_h.at[pl.ds(out_off, _RPT)])

    return k(P, Q, V, RB1f, consts, esrc, edst, ehop, zrows)


def kernel(all_emb0, all_emb1, all_emb2, Wf0, Wf1, Wf2, W_ext1, b_ext1,
           W_ext2, b_ext2, W_enc1, W_enc2, W_out, node_ids, edge_src,
           edge_dst, e_hop, labels):
    W1a = W_ext1[0:_HID]
    W1b = W_ext1[_HID:2 * _HID]
    W1c = W_ext1[2 * _HID:3 * _HID]
    consts = jnp.concatenate([W_ext2[:, 0], b_ext2,
                              jnp.zeros((15,), jnp.float32)])
    zeros = jnp.zeros((_N, _ENC), jnp.float32)

    G0, G1, G2 = _gather3(all_emb0, all_emb1, all_emb2, node_ids)
    P0, Q0, P1, Q1, XW, F2 = _dense_node(
        G0, G1, G2, Wf0, Wf1, Wf2, W1a, W1b, W_enc1)
    F2o = F2.reshape(_B, _NPG, _HID)[:, 0, :]
    RB1 = _rb_tc(F2o, W1c, b_ext1.reshape(1, _HID))
    RB1f = RB1.reshape(-1)

    H1P = _edge_pass(P0, Q0, XW, RB1f, consts, edge_src, edge_dst, e_hop,
                     zeros, hop_sel=1)
    HW1 = _dense_mid(H1P.reshape(_NC, _N, _ENC), W_enc2)
    H2P = _edge_pass(P1, Q1, HW1, RB1f, consts, edge_src, edge_dst, e_hop,
                     zeros, hop_sel=0)
    H2o = H2P.reshape(_NC, _B, _NPG, _ENC)[:, :, 0, :]
    return _final_tc(H2o, W_out)


# speedup vs baseline: 9.9045x; 1.4827x over previous
import functools

import jax
import jax.numpy as jnp
from jax import lax
from jax.experimental import pallas as pl
from jax.experimental.pallas import tpu as pltpu
from jax.experimental.pallas import tpu_sc as plsc

_N_TOTAL = 50000
_N = 10000
_E = 160000
_B = 16
_NPG = 625
_IN = 128
_HID = 32
_ENC = 64
_NC = 2
_NS = 16
_NW = _NC * _NS
_EPW = _E // _NW
_CHUNK = 128
_NFULL = _EPW // _CHUNK
_TAIL = _EPW - _NFULL * _CHUNK
_RPT = _N // _NS
_GPW = 312
_GCH = 104


def _sc_mesh():
    return plsc.VectorSubcoreMesh(
        core_axis_name="c", subcore_axis_name="s",
        num_cores=_NC, num_subcores=_NS)


_SC_PARAMS = pltpu.CompilerParams(use_tc_tiling_on_sc=False,
                                  needs_layout_passes=False)


def _gather_rows(tables, nid, params):
    n_t = len(tables)
    widths = [t.shape[1] for t in tables]

    @functools.partial(
        pl.kernel,
        out_type=tuple(
            jax.ShapeDtypeStruct((_N, w), jnp.float32) for w in widths),
        mesh=_sc_mesh(),
        scratch_types=(
            [pltpu.VMEM((_GCH,), jnp.int32),
             pltpu.VMEM((16,), jnp.int32)]
            + [pltpu.VMEM((_GCH, w), jnp.float32) for w in widths]
            + [pltpu.VMEM((16, w), jnp.float32) for w in widths]
            + [pltpu.SemaphoreType.DMA]
        ),
        compiler_params=params,
    )
    def k(*refs):
        embs = refs[:n_t]
        nid_h = refs[n_t]
        outs = refs[n_t + 1:2 * n_t + 1]
        idx_v, idx_t = refs[2 * n_t + 1:2 * n_t + 3]
        bufs = refs[2 * n_t + 3:3 * n_t + 3]
        tbufs = refs[3 * n_t + 3:4 * n_t + 3]
        sem = refs[-1]
        wid = lax.axis_index("s") * _NC + lax.axis_index("c")
        base = wid * _GPW
        for ci in range(_GPW // _GCH):
            off = base + ci * _GCH
            pltpu.sync_copy(nid_h.at[pl.ds(off, _GCH)], idx_v)
            cps = [pltpu.async_copy(e.at[idx_v], b, sem)
                   for e, b in zip(embs, bufs)]
            for c in cps:
                c.wait()
            for b, o in zip(bufs, outs):
                pltpu.sync_copy(b, o.at[pl.ds(off, _GCH)])

        @pl.when(wid == 0)
        def _tail():
            off = _NW * _GPW
            pltpu.sync_copy(nid_h.at[pl.ds(off, 16)], idx_t)
            cps = [pltpu.async_copy(e.at[idx_t], b, sem)
                   for e, b in zip(embs, tbufs)]
            for c in cps:
                c.wait()
            for b, o in zip(tbufs, outs):
                pltpu.sync_copy(b, o.at[pl.ds(off, 16)])

    return k(*tables, nid)


def _dense_node(G0, G1, G2, Wf0, Wf1, Wf2, W1a, W1b, Wenc1):
    blk = 1000

    def body(g0, g1, g2, wf0, wf1, wf2, w1a, w1b, we1,
             p0, q0, p1, q1, xw, f2o):
        f0 = jnp.maximum(jnp.dot(g0[...], wf0[...],
                                 preferred_element_type=jnp.float32), 0.0)
        f1 = jnp.maximum(jnp.dot(g1[...], wf1[...],
                                 preferred_element_type=jnp.float32), 0.0)
        f2 = jnp.maximum(jnp.dot(g2[...], wf2[...],
                                 preferred_element_type=jnp.float32), 0.0)
        p0[...] = jnp.dot(f0, w1a[...], preferred_element_type=jnp.float32)
        q0[...] = jnp.dot(f1, w1b[...], preferred_element_type=jnp.float32)
        p1[...] = jnp.dot(f1, w1a[...], preferred_element_type=jnp.float32)
        q1[...] = jnp.dot(f2, w1b[...], preferred_element_type=jnp.float32)
        xw[...] = jnp.dot(g0[...], we1[...],
                          preferred_element_type=jnp.float32)
        f2o[...] = f2

    full = lambda a, b: pl.BlockSpec((a, b), lambda i: (0, 0))
    row = lambda w: pl.BlockSpec((blk, w), lambda i: (i, 0))
    outs = [jax.ShapeDtypeStruct((_N, _HID), jnp.float32)] * 4 + [
        jax.ShapeDtypeStruct((_N, _ENC), jnp.float32),
        jax.ShapeDtypeStruct((_N, _HID), jnp.float32)]
    return pl.pallas_call(
        body,
        grid=(_N // blk,),
        in_specs=[row(128), row(256), row(64),
                  full(128, _HID), full(256, _HID), full(64, _HID),
                  full(_HID, _HID), full(_HID, _HID), full(128, _ENC)],
        out_specs=[row(_HID), row(_HID), row(_HID), row(_HID),
                   row(_ENC), row(_HID)],
        out_shape=outs,
    )(G0, G1, G2, Wf0, Wf1, Wf2, W1a, W1b, Wenc1)


def _rb_tc(F2o, W1c, b1r):
    def body(f, w, b, o):
        o[...] = jnp.dot(f[...], w[...],
                         preferred_element_type=jnp.float32) + b[...]

    return pl.pallas_call(
        body,
        out_shape=jax.ShapeDtypeStruct((_B, _HID), jnp.float32),
    )(F2o, W1c, b1r)


def _dense_mid(HP, W):
    blk = 1000

    def body(hp, w, o):
        h = jnp.maximum(hp[0] + hp[1], 0.0)
        o[...] = jnp.dot(h, w[...], preferred_element_type=jnp.float32)

    return pl.pallas_call(
        body,
        grid=(_N // blk,),
        in_specs=[pl.BlockSpec((2, blk, _ENC), lambda i: (0, i, 0)),
                  pl.BlockSpec((_ENC, _ENC), lambda i: (0, 0))],
        out_specs=pl.BlockSpec((blk, _ENC), lambda i: (i, 0)),
        out_shape=jax.ShapeDtypeStruct((_N, _ENC), jnp.float32),
    )(HP, W)


def _final_tc(H2o, Wout):
    def body(h, w, o):
        o[...] = jnp.dot(h[0] + h[1], w[...],
                         preferred_element_type=jnp.float32)

    return pl.pallas_call(
        body,
        out_shape=jax.ShapeDtypeStruct((_B, 8), jnp.float32),
    )(H2o, Wout)


def _edge_pass(P, Q, V, RB1f, consts, esrc, edst, ehop, zrows, hop_sel,
               sparse_out):
    out_rows = _NC * _B if sparse_out else _NC * _N

    @functools.partial(
        pl.kernel,
        out_type=jax.ShapeDtypeStruct((out_rows, _ENC), jnp.float32),
        mesh=_sc_mesh(),
        scratch_types=[
            [pltpu.VMEM((_CHUNK,), jnp.int32)] * 2,
            [pltpu.VMEM((_CHUNK,), jnp.int32)] * 2,
            [pltpu.VMEM((_CHUNK,), jnp.int32)] * 2,
            pltpu.VMEM((16,), jnp.int32),
            pltpu.VMEM((16,), jnp.int32),
            pltpu.VMEM((16,), jnp.int32),
            [pltpu.VMEM((_CHUNK, _HID), jnp.float32)] * 2,
            [pltpu.VMEM((_CHUNK, _HID), jnp.float32)] * 2,
            [pltpu.VMEM((_CHUNK, _ENC), jnp.float32)] * 2,
            pltpu.VMEM((_CHUNK, _ENC), jnp.float32),
            pltpu.VMEM((16, _HID), jnp.float32),
            pltpu.VMEM((16, _HID), jnp.float32),
            pltpu.VMEM((16, _ENC), jnp.float32),
            pltpu.VMEM((16, _ENC), jnp.float32),
            pltpu.VMEM((512,), jnp.float32),
            pltpu.VMEM((512,), jnp.float32),
            pltpu.VMEM((48,), jnp.float32),
            pltpu.VMEM_SHARED((_N, _ENC), jnp.float32),
            [pltpu.SemaphoreType.DMA] * 2,
            [pltpu.SemaphoreType.DMA] * 2,
        ],
        compiler_params=_SC_PARAMS,
    )
    def k(P_h, Q_h, V_h, rb_h, c_h, es_h, ed_h, eh_h, z_h, out_h,
          sidx, didx, hbuf, sidx16, didx16, hidx16,
          pbuf, qbuf, vbuf, mbuf, pbuf16, qbuf16, vbuf16, mbuf16,
          tt, rbv, cv, hacc, isem, gsem):
        cid = lax.axis_index("c")
        sid = lax.axis_index("s")
        wid = sid * _NC + cid
        if sparse_out:
            pltpu.sync_copy(z_h.at[pl.ds(0, 1)],
                            hacc.at[pl.ds(sid * _NPG, 1)])
        else:
            pltpu.sync_copy(z_h.at[pl.ds(sid * _RPT, _RPT)],
                            hacc.at[pl.ds(sid * _RPT, _RPT)])
        pltpu.sync_copy(rb_h, rbv)
        pltpu.sync_copy(c_h, cv)
        plsc.subcore_barrier()

        lane = lax.iota(jnp.int32, 16)
        b = wid // 2
        rb_lo = rbv[pl.ds(pl.multiple_of(b * 32, 16), 16)]
        rb_hi = rbv[pl.ds(pl.multiple_of(b * 32 + 16, 16), 16)]
        w2_lo = cv[pl.ds(0, 16)]
        w2_hi = cv[pl.ds(16, 16)]
        b2s = cv[pl.ds(32, 16)][0]
        ebase = wid * _EPW

        def iget(ci_next, nb):
            nxt = jnp.minimum(ci_next, _NFULL - 1)
            nbase = pl.multiple_of(ebase + nxt * _CHUNK, 8)
            pltpu.async_copy(es_h.at[pl.ds(nbase, _CHUNK)], sidx[nb],
                             isem[nb])
            pltpu.async_copy(ed_h.at[pl.ds(nbase, _CHUNK)], didx[nb],
                             isem[nb])
            pltpu.async_copy(eh_h.at[pl.ds(nbase, _CHUNK)], hbuf[nb],
                             isem[nb])

        def iwait(nb):
            pltpu.make_async_copy(es_h.at[pl.ds(0, _CHUNK)], sidx[nb],
                                  isem[nb]).wait()
            pltpu.make_async_copy(ed_h.at[pl.ds(0, _CHUNK)], didx[nb],
                                  isem[nb]).wait()
            pltpu.make_async_copy(eh_h.at[pl.ds(0, _CHUNK)], hbuf[nb],
                                  isem[nb]).wait()

        def g_issue(bb):
            pltpu.async_copy(P_h.at[sidx[bb]], pbuf[bb], gsem[bb])
            pltpu.async_copy(Q_h.at[didx[bb]], qbuf[bb], gsem[bb])
            pltpu.async_copy(V_h.at[sidx[bb]], vbuf[bb], gsem[bb])

        def g_wait(bb):
            pltpu.make_async_copy(P_h.at[sidx[bb]], pbuf[bb],
                                  gsem[bb]).wait()
            pltpu.make_async_copy(Q_h.at[didx[bb]], qbuf[bb],
                                  gsem[bb]).wait()
            pltpu.make_async_copy(V_h.at[sidx[bb]], vbuf[bb],
                                  gsem[bb]).wait()

        def active(bb):
            m = (didx[bb][pl.ds(0, 16)] % _NPG) == 0
            for jj in range(1, _CHUNK // 16):
                m = m | ((didx[bb][pl.ds(jj * 16, 16)] % _NPG) == 0)
            return jnp.any(m)

        def group(pb, qb, vb, mb, hv, e0):
            for le in range(16):
                e = e0 + le
                p_lo = pb[e, pl.ds(0, 16)]
                p_hi = pb[e, pl.ds(16, 16)]
                q_lo = qb[e, pl.ds(0, 16)]
                q_hi = qb[e, pl.ds(16, 16)]
                t_lo = jnp.maximum(p_lo + q_lo + rb_lo, 0.0)
                t_hi = jnp.maximum(p_hi + q_hi + rb_hi, 0.0)
                plsc.store_scatter(tt, [lane * 16 + le], t_lo)
                plsc.store_scatter(tt, [lane * 16 + (256 + le)], t_hi)
            att = jnp.full((16,), 0.0, jnp.float32) + b2s
            for kk in range(16):
                att = att + tt[pl.ds(kk * 16, 16)] * w2_lo[kk]
            for kk in range(16):
                att = att + tt[pl.ds((kk + 16) * 16, 16)] * w2_hi[kk]
            sig = 1.0 / (1.0 + jnp.exp(-att))
            gate = jnp.where(hv == hop_sel, sig, 0.0)
            for le in range(16):
                e = e0 + le
                g_s = gate[le]
                for j in range(_ENC // 16):
                    mb[e, pl.ds(j * 16, 16)] = vb[e, pl.ds(j * 16, 16)] * g_s

        def compute_scatter(bb):
            def group_body(g, __):
                e0 = g * 16
                hv = hbuf[bb][pl.ds(e0, 16)]
                group(pbuf[bb], qbuf[bb], vbuf[bb], mbuf, hv, e0)
                return __

            lax.fori_loop(0, _CHUNK // 16, group_body, 0)
            pltpu.sync_copy(mbuf, hacc.at[didx[bb]], add=True)

        def step(ci, bb):
            nb = 1 - bb
            if sparse_out:
                act = active(bb)

                @pl.when(act)
                def _():
                    g_wait(bb)
            else:
                g_wait(bb)
            iwait(nb)
            if sparse_out:
                @pl.when(active(nb))
                def _():
                    g_issue(nb)
            else:
                g_issue(nb)
            if sparse_out:
                @pl.when(act)
                def _():
                    compute_scatter(bb)
            else:
                compute_scatter(bb)
            iget(ci + 2, bb)

        iget(0, 0)
        iwait(0)
        if sparse_out:
            @pl.when(active(0))
            def _():
                g_issue(0)
        else:
            g_issue(0)
        iget(1, 1)

        def pair_body(i, _):
            step(2 * i, 0)
            step(2 * i + 1, 1)
            return _

        lax.fori_loop(0, (_NFULL - 1) // 2, pair_body, 0)
        step(_NFULL - 1, 0)
        if sparse_out:
            @pl.when(active(1))
            def _():
                g_wait(1)
        else:
            g_wait(1)
        iwait(0)

        tbase = ebase + _NFULL * _CHUNK
        pltpu.sync_copy(es_h.at[pl.ds(tbase, 8)], sidx[0].at[pl.ds(0, 8)])
        pltpu.sync_copy(ed_h.at[pl.ds(tbase, 8)], didx[0].at[pl.ds(0, 8)])
        pltpu.sync_copy(eh_h.at[pl.ds(tbase, 8)], hbuf[0].at[pl.ds(0, 8)])
        pad = lane < 8
        sidx16[...] = jnp.where(pad, sidx[0][pl.ds(0, 16)], 1)
        didx16[...] = jnp.where(pad, didx[0][pl.ds(0, 16)], 1)
        hidx16[...] = jnp.where(pad, hbuf[0][pl.ds(0, 16)], -1)
        cp = pltpu.async_copy(P_h.at[sidx16], pbuf16, gsem[0])
        cq = pltpu.async_copy(Q_h.at[didx16], qbuf16, gsem[0])
        cvv = pltpu.async_copy(V_h.at[sidx16], vbuf16, gsem[0])
        cp.wait()
        cq.wait()
        cvv.wait()
        group(pbuf16, qbuf16, vbuf16, mbuf16, hidx16[...], 0)
        pltpu.sync_copy(mbuf16, hacc.at[didx16], add=True)

        plsc.subcore_barrier()
        if sparse_out:
            pltpu.sync_copy(hacc.at[pl.ds(sid * _NPG, 1)],
                            out_h.at[pl.ds(cid * _B + sid, 1)])
        else:
            out_off = pl.multiple_of(cid * _N + sid * _RPT, 8)
            pltpu.sync_copy(hacc.at[pl.ds(sid * _RPT, _RPT)],
                            out_h.at[pl.ds(out_off, _RPT)])

    return k(P, Q, V, RB1f, consts, esrc, edst, ehop, zrows)


def kernel(all_emb0, all_emb1, all_emb2, Wf0, Wf1, Wf2, W_ext1, b_ext1,
           W_ext2, b_ext2, W_enc1, W_enc2, W_out, node_ids, edge_src,
           edge_dst, e_hop, labels):
    W1a = W_ext1[0:_HID]
    W1b = W_ext1[_HID:2 * _HID]
    W1c = W_ext1[2 * _HID:3 * _HID]
    consts = jnp.concatenate([W_ext2[:, 0], b_ext2,
                              jnp.zeros((15,), jnp.float32)])
    zeros = jnp.zeros((_N, _ENC), jnp.float32)

    G0, G1 = _gather_rows((all_emb0, all_emb1), node_ids, None)
    (G2,) = _gather_rows((all_emb2,), node_ids, _SC_PARAMS)
    P0, Q0, P1, Q1, XW, F2 = _dense_node(
        G0, G1, G2, Wf0, Wf1, Wf2, W1a, W1b, W_enc1)
    F2o = F2.reshape(_B, _NPG, _HID)[:, 0, :]
    RB1 = _rb_tc(F2o, W1c, b_ext1.reshape(1, _HID))
    RB1f = RB1.reshape(-1)

    H1P = _edge_pass(P0, Q0, XW, RB1f, consts, edge_src, edge_dst, e_hop,
                     zeros, hop_sel=1, sparse_out=False)
    HW1 = _dense_mid(H1P.reshape(_NC, _N, _ENC), W_enc2)
    H2P = _edge_pass(P1, Q1, HW1, RB1f, consts, edge_src, edge_dst, e_hop,
                     zeros, hop_sel=0, sparse_out=True)
    return _final_tc(H2P.reshape(_NC, _B, _ENC), W_out)


# speedup vs baseline: 10.4178x; 1.0518x over previous
import functools

import jax
import jax.numpy as jnp
from jax import lax
from jax.experimental import pallas as pl
from jax.experimental.pallas import tpu as pltpu
from jax.experimental.pallas import tpu_sc as plsc

_N_TOTAL = 50000
_N = 10000
_E = 160000
_B = 16
_NPG = 625
_IN = 128
_HID = 32
_ENC = 64
_NC = 2
_NS = 16
_NW = _NC * _NS
_EPW = _E // _NW
_CHUNK = 128
_NFULL = _EPW // _CHUNK
_TAIL = _EPW - _NFULL * _CHUNK
_RPT = _N // _NS
_GPW = 312
_GCH = 104


def _sc_mesh():
    return plsc.VectorSubcoreMesh(
        core_axis_name="c", subcore_axis_name="s",
        num_cores=_NC, num_subcores=_NS)


_SC_PARAMS = pltpu.CompilerParams(use_tc_tiling_on_sc=False,
                                  needs_layout_passes=False)


def _gather_rows(tables, nid, params):
    n_t = len(tables)
    widths = [t.shape[1] for t in tables]

    @functools.partial(
        pl.kernel,
        out_type=tuple(
            jax.ShapeDtypeStruct((_N, w), jnp.float32) for w in widths),
        mesh=_sc_mesh(),
        scratch_types=(
            [pltpu.VMEM((_GCH,), jnp.int32),
             pltpu.VMEM((16,), jnp.int32)]
            + [pltpu.VMEM((_GCH, w), jnp.float32) for w in widths]
            + [pltpu.VMEM((16, w), jnp.float32) for w in widths]
            + [pltpu.SemaphoreType.DMA]
        ),
        compiler_params=params,
    )
    def k(*refs):
        embs = refs[:n_t]
        nid_h = refs[n_t]
        outs = refs[n_t + 1:2 * n_t + 1]
        idx_v, idx_t = refs[2 * n_t + 1:2 * n_t + 3]
        bufs = refs[2 * n_t + 3:3 * n_t + 3]
        tbufs = refs[3 * n_t + 3:4 * n_t + 3]
        sem = refs[-1]
        wid = lax.axis_index("s") * _NC + lax.axis_index("c")
        base = wid * _GPW
        for ci in range(_GPW // _GCH):
            off = base + ci * _GCH
            pltpu.sync_copy(nid_h.at[pl.ds(off, _GCH)], idx_v)
            cps = [pltpu.async_copy(e.at[idx_v], b, sem)
                   for e, b in zip(embs, bufs)]
            for c in cps:
                c.wait()
            for b, o in zip(bufs, outs):
                pltpu.sync_copy(b, o.at[pl.ds(off, _GCH)])

        @pl.when(wid == 0)
        def _tail():
            off = _NW * _GPW
            pltpu.sync_copy(nid_h.at[pl.ds(off, 16)], idx_t)
            cps = [pltpu.async_copy(e.at[idx_t], b, sem)
                   for e, b in zip(embs, tbufs)]
            for c in cps:
                c.wait()
            for b, o in zip(tbufs, outs):
                pltpu.sync_copy(b, o.at[pl.ds(off, 16)])

    return k(*tables, nid)


def _dense_node(G0, G1, G2, Wf0, Wf1, Wf2, W1a, W1b, Wenc1):
    blk = 1000

    def body(g0, g1, g2, wf0, wf1, wf2, w1a, w1b, we1,
             p0, q0, p1, q1, xw, f2o):
        f0 = jnp.maximum(jnp.dot(g0[...], wf0[...],
                                 preferred_element_type=jnp.float32), 0.0)
        f1 = jnp.maximum(jnp.dot(g1[...], wf1[...],
                                 preferred_element_type=jnp.float32), 0.0)
        f2 = jnp.maximum(jnp.dot(g2[...], wf2[...],
                                 preferred_element_type=jnp.float32), 0.0)
        p0[...] = jnp.dot(f0, w1a[...], preferred_element_type=jnp.float32)
        q0[...] = jnp.dot(f1, w1b[...], preferred_element_type=jnp.float32)
        p1[...] = jnp.dot(f1, w1a[...], preferred_element_type=jnp.float32)
        q1[...] = jnp.dot(f2, w1b[...], preferred_element_type=jnp.float32)
        xw[...] = jnp.dot(g0[...], we1[...],
                          preferred_element_type=jnp.float32)
        f2o[...] = f2

    full = lambda a, b: pl.BlockSpec((a, b), lambda i: (0, 0))
    row = lambda w: pl.BlockSpec((blk, w), lambda i: (i, 0))
    outs = [jax.ShapeDtypeStruct((_N, _HID), jnp.float32)] * 4 + [
        jax.ShapeDtypeStruct((_N, _ENC), jnp.float32),
        jax.ShapeDtypeStruct((_N, _HID), jnp.float32)]
    return pl.pallas_call(
        body,
        grid=(_N // blk,),
        in_specs=[row(128), row(256), row(64),
                  full(128, _HID), full(256, _HID), full(64, _HID),
                  full(_HID, _HID), full(_HID, _HID), full(128, _ENC)],
        out_specs=[row(_HID), row(_HID), row(_HID), row(_HID),
                   row(_ENC), row(_HID)],
        out_shape=outs,
    )(G0, G1, G2, Wf0, Wf1, Wf2, W1a, W1b, Wenc1)


def _rb_tc(F2o, W1c, b1r):
    def body(f, w, b, o):
        o[...] = jnp.dot(f[...], w[...],
                         preferred_element_type=jnp.float32) + b[...]

    return pl.pallas_call(
        body,
        out_shape=jax.ShapeDtypeStruct((_B, _HID), jnp.float32),
    )(F2o, W1c, b1r)


def _dense_mid(HP, W):
    blk = 1000

    def body(hp, w, o):
        h = jnp.maximum(hp[0] + hp[1], 0.0)
        o[...] = jnp.dot(h, w[...], preferred_element_type=jnp.float32)

    return pl.pallas_call(
        body,
        grid=(_N // blk,),
        in_specs=[pl.BlockSpec((2, blk, _ENC), lambda i: (0, i, 0)),
                  pl.BlockSpec((_ENC, _ENC), lambda i: (0, 0))],
        out_specs=pl.BlockSpec((blk, _ENC), lambda i: (i, 0)),
        out_shape=jax.ShapeDtypeStruct((_N, _ENC), jnp.float32),
    )(HP, W)


def _final_tc(H2o, Wout):
    def body(h, w, o):
        o[...] = jnp.dot(h[0] + h[1], w[...],
                         preferred_element_type=jnp.float32)

    return pl.pallas_call(
        body,
        out_shape=jax.ShapeDtypeStruct((_B, 8), jnp.float32),
    )(H2o, Wout)


def _edge_pass(P, Q, V, RB1f, consts, esrc, edst, ehop, zrows, hop_sel,
               sparse_out, flags=None):
    if sparse_out:
        out_type = jax.ShapeDtypeStruct((_NC * _B, _ENC), jnp.float32)
        extra_in = (flags,)
    else:
        out_type = (jax.ShapeDtypeStruct((_NC * _N, _ENC), jnp.float32),
                    jax.ShapeDtypeStruct((_NW * 40,), jnp.int32))
        extra_in = ()

    @functools.partial(
        pl.kernel,
        out_type=out_type,
        mesh=_sc_mesh(),
        scratch_types=[
            [pltpu.VMEM((_CHUNK,), jnp.int32)] * 2,
            [pltpu.VMEM((_CHUNK,), jnp.int32)] * 2,
            [pltpu.VMEM((_CHUNK,), jnp.int32)] * 2,
            pltpu.VMEM((16,), jnp.int32),
            pltpu.VMEM((16,), jnp.int32),
            pltpu.VMEM((16,), jnp.int32),
            [pltpu.VMEM((_CHUNK, _HID), jnp.float32)] * 2,
            [pltpu.VMEM((_CHUNK, _HID), jnp.float32)] * 2,
            [pltpu.VMEM((_CHUNK, _ENC), jnp.float32)] * 2,
            pltpu.VMEM((_CHUNK, _ENC), jnp.float32),
            pltpu.VMEM((16, _HID), jnp.float32),
            pltpu.VMEM((16, _HID), jnp.float32),
            pltpu.VMEM((16, _ENC), jnp.float32),
            pltpu.VMEM((16, _ENC), jnp.float32),
            pltpu.VMEM((512,), jnp.float32),
            pltpu.VMEM((512,), jnp.float32),
            pltpu.VMEM((48,), jnp.float32),
            pltpu.VMEM((40,), jnp.int32),
            pltpu.VMEM_SHARED((_N, _ENC), jnp.float32),
            [pltpu.SemaphoreType.DMA] * 2,
            [pltpu.SemaphoreType.DMA] * 2,
        ],
        compiler_params=_SC_PARAMS,
    )
    def k(P_h, Q_h, V_h, rb_h, c_h, es_h, ed_h, eh_h, z_h, *refs):
        if sparse_out:
            flags_h, out_h = refs[0], refs[1]
        else:
            out_h, flags_o = refs[0], refs[1]
        (sidx, didx, hbuf, sidx16, didx16, hidx16,
         pbuf, qbuf, vbuf, mbuf, pbuf16, qbuf16, vbuf16, mbuf16,
         tt, rbv, cv, fbuf, hacc, isem, gsem) = refs[2:]
        cid = lax.axis_index("c")
        sid = lax.axis_index("s")
        wid = sid * _NC + cid
        if sparse_out:
            pltpu.sync_copy(z_h.at[pl.ds(0, 1)],
                            hacc.at[pl.ds(sid * _NPG, 1)])
        else:
            pltpu.sync_copy(z_h.at[pl.ds(sid * _RPT, _RPT)],
                            hacc.at[pl.ds(sid * _RPT, _RPT)])
        pltpu.sync_copy(rb_h, rbv)
        pltpu.sync_copy(c_h, cv)
        plsc.subcore_barrier()

        lane = lax.iota(jnp.int32, 16)
        b = wid // 2
        rb_lo = rbv[pl.ds(pl.multiple_of(b * 32, 16), 16)]
        rb_hi = rbv[pl.ds(pl.multiple_of(b * 32 + 16, 16), 16)]
        w2_lo = cv[pl.ds(0, 16)]
        w2_hi = cv[pl.ds(16, 16)]
        b2s = cv[pl.ds(32, 16)][0]
        ebase = wid * _EPW

        def iget(ci_next, nb):
            nxt = jnp.minimum(ci_next, _NFULL - 1)
            nbase = pl.multiple_of(ebase + nxt * _CHUNK, 8)
            pltpu.async_copy(es_h.at[pl.ds(nbase, _CHUNK)], sidx[nb],
                             isem[nb])
            pltpu.async_copy(ed_h.at[pl.ds(nbase, _CHUNK)], didx[nb],
                             isem[nb])
            pltpu.async_copy(eh_h.at[pl.ds(nbase, _CHUNK)], hbuf[nb],
                             isem[nb])

        def iwait(nb):
            pltpu.make_async_copy(es_h.at[pl.ds(0, _CHUNK)], sidx[nb],
                                  isem[nb]).wait()
            pltpu.make_async_copy(ed_h.at[pl.ds(0, _CHUNK)], didx[nb],
                                  isem[nb]).wait()
            pltpu.make_async_copy(eh_h.at[pl.ds(0, _CHUNK)], hbuf[nb],
                                  isem[nb]).wait()

        def g_issue(bb):
            pltpu.async_copy(P_h.at[sidx[bb]], pbuf[bb], gsem[bb])
            pltpu.async_copy(Q_h.at[didx[bb]], qbuf[bb], gsem[bb])
            pltpu.async_copy(V_h.at[sidx[bb]], vbuf[bb], gsem[bb])

        def g_wait(bb):
            pltpu.make_async_copy(P_h.at[sidx[bb]], pbuf[bb],
                                  gsem[bb]).wait()
            pltpu.make_async_copy(Q_h.at[didx[bb]], qbuf[bb],
                                  gsem[bb]).wait()
            pltpu.make_async_copy(V_h.at[sidx[bb]], vbuf[bb],
                                  gsem[bb]).wait()

        def active(bb):
            m = (didx[bb][pl.ds(0, 16)] % _NPG) == 0
            for jj in range(1, _CHUNK // 16):
                m = m | ((didx[bb][pl.ds(jj * 16, 16)] % _NPG) == 0)
            return jnp.any(m)

        def group(pb, qb, vb, mb, hv, e0):
            for le in range(16):
                e = e0 + le
                p_lo = pb[e, pl.ds(0, 16)]
                p_hi = pb[e, pl.ds(16, 16)]
                q_lo = qb[e, pl.ds(0, 16)]
                q_hi = qb[e, pl.ds(16, 16)]
                t_lo = jnp.maximum(p_lo + q_lo + rb_lo, 0.0)
                t_hi = jnp.maximum(p_hi + q_hi + rb_hi, 0.0)
                plsc.store_scatter(tt, [lane * 16 + le], t_lo)
                plsc.store_scatter(tt, [lane * 16 + (256 + le)], t_hi)
            att = jnp.full((16,), 0.0, jnp.float32) + b2s
            for kk in range(16):
                att = att + tt[pl.ds(kk * 16, 16)] * w2_lo[kk]
            for kk in range(16):
                att = att + tt[pl.ds((kk + 16) * 16, 16)] * w2_hi[kk]
            sig = 1.0 / (1.0 + jnp.exp(-att))
            gate = jnp.where(hv == hop_sel, sig, 0.0)
            for le in range(16):
                e = e0 + le
                g_s = gate[le]
                for j in range(_ENC // 16):
                    mb[e, pl.ds(j * 16, 16)] = vb[e, pl.ds(j * 16, 16)] * g_s

        def compute_scatter(bb):
            def group_body(g, __):
                e0 = g * 16
                hv = hbuf[bb][pl.ds(e0, 16)]
                group(pbuf[bb], qbuf[bb], vbuf[bb], mbuf, hv, e0)
                return __

            lax.fori_loop(0, _CHUNK // 16, group_body, 0)
            pltpu.sync_copy(mbuf, hacc.at[didx[bb]], add=True)

        if sparse_out:
            pltpu.sync_copy(flags_h.at[pl.ds(wid * 40, 40)], fbuf)

            def chunk_body(ci, _):
                fl = plsc.load_gather(
                    fbuf, [jnp.full((16,), ci, jnp.int32)])

                @pl.when(jnp.any(fl != 0))
                def _a():
                    base = pl.multiple_of(ebase + ci * _CHUNK, 8)
                    pltpu.async_copy(es_h.at[pl.ds(base, _CHUNK)],
                                     sidx[0], isem[0])
                    pltpu.async_copy(ed_h.at[pl.ds(base, _CHUNK)],
                                     didx[0], isem[0])
                    pltpu.async_copy(eh_h.at[pl.ds(base, _CHUNK)],
                                     hbuf[0], isem[0])
                    iwait(0)
                    g_issue(0)
                    g_wait(0)
                    compute_scatter(0)

                return _

            lax.fori_loop(0, _NFULL, chunk_body, 0)
        else:
            def step(ci, bb):
                nb = 1 - bb
                g_wait(bb)
                iwait(nb)
                g_issue(nb)
                avec = jnp.where(jnp.full((16,), active(bb)),
                                 jnp.ones((16,), jnp.int32),
                                 jnp.zeros((16,), jnp.int32))
                plsc.store_scatter(fbuf,
                                   [jnp.full((16,), ci, jnp.int32)],
                                   avec, mask=lane < 1)
                compute_scatter(bb)
                iget(ci + 2, bb)

            iget(0, 0)
            iwait(0)
            g_issue(0)
            iget(1, 1)

            def pair_body(i, _):
                step(2 * i, 0)
                step(2 * i + 1, 1)
                return _

            lax.fori_loop(0, (_NFULL - 1) // 2, pair_body, 0)
            step(_NFULL - 1, 0)
            g_wait(1)
            iwait(0)
            pltpu.sync_copy(fbuf, flags_o.at[pl.ds(wid * 40, 40)])

        tbase = ebase + _NFULL * _CHUNK
        pltpu.sync_copy(es_h.at[pl.ds(tbase, 8)], sidx[0].at[pl.ds(0, 8)])
        pltpu.sync_copy(ed_h.at[pl.ds(tbase, 8)], didx[0].at[pl.ds(0, 8)])
        pltpu.sync_copy(eh_h.at[pl.ds(tbase, 8)], hbuf[0].at[pl.ds(0, 8)])
        pad = lane < 8
        sidx16[...] = jnp.where(pad, sidx[0][pl.ds(0, 16)], 1)
        didx16[...] = jnp.where(pad, didx[0][pl.ds(0, 16)], 1)
        hidx16[...] = jnp.where(pad, hbuf[0][pl.ds(0, 16)], -1)
        cp = pltpu.async_copy(P_h.at[sidx16], pbuf16, gsem[0])
        cq = pltpu.async_copy(Q_h.at[didx16], qbuf16, gsem[0])
        cvv = pltpu.async_copy(V_h.at[sidx16], vbuf16, gsem[0])
        cp.wait()
        cq.wait()
        cvv.wait()
        group(pbuf16, qbuf16, vbuf16, mbuf16, hidx16[...], 0)
        pltpu.sync_copy(mbuf16, hacc.at[didx16], add=True)

        plsc.subcore_barrier()
        if sparse_out:
            pltpu.sync_copy(hacc.at[pl.ds(sid * _NPG, 1)],
                            out_h.at[pl.ds(cid * _B + sid, 1)])
        else:
            out_off = pl.multiple_of(cid * _N + sid * _RPT, 8)
            pltpu.sync_copy(hacc.at[pl.ds(sid * _RPT, _RPT)],
                            out_h.at[pl.ds(out_off, _RPT)])

    return k(P, Q, V, RB1f, consts, esrc, edst, ehop, zrows, *extra_in)


def kernel(all_emb0, all_emb1, all_emb2, Wf0, Wf1, Wf2, W_ext1, b_ext1,
           W_ext2, b_ext2, W_enc1, W_enc2, W_out, node_ids, edge_src,
           edge_dst, e_hop, labels):
    W1a = W_ext1[0:_HID]
    W1b = W_ext1[_HID:2 * _HID]
    W1c = W_ext1[2 * _HID:3 * _HID]
    consts = jnp.concatenate([W_ext2[:, 0], b_ext2,
                              jnp.zeros((15,), jnp.float32)])
    zeros = jnp.zeros((_N, _ENC), jnp.float32)

    G0, G1 = _gather_rows((all_emb0, all_emb1), node_ids, None)
    (G2,) = _gather_rows((all_emb2,), node_ids, _SC_PARAMS)
    P0, Q0, P1, Q1, XW, F2 = _dense_node(
        G0, G1, G2, Wf0, Wf1, Wf2, W1a, W1b, W_enc1)
    F2o = F2.reshape(_B, _NPG, _HID)[:, 0, :]
    RB1 = _rb_tc(F2o, W1c, b_ext1.reshape(1, _HID))
    RB1f = RB1.reshape(-1)

    H1P, FLG = _edge_pass(P0, Q0, XW, RB1f, consts, edge_src, edge_dst,
                          e_hop, zeros, hop_sel=1, sparse_out=False)
    HW1 = _dense_mid(H1P.reshape(_NC, _N, _ENC), W_enc2)
    H2P = _edge_pass(P1, Q1, HW1, RB1f, consts, edge_src, edge_dst, e_hop,
                     zeros, hop_sel=0, sparse_out=True, flags=FLG)
    return _final_tc(H2P.reshape(_NC, _B, _ENC), W_out)


# speedup vs baseline: 11.1869x; 1.0738x over previous
import functools

import jax
import jax.numpy as jnp
from jax import lax
from jax.experimental import pallas as pl
from jax.experimental.pallas import tpu as pltpu
from jax.experimental.pallas import tpu_sc as plsc

_N_TOTAL = 50000
_N = 10000
_E = 160000
_B = 16
_NPG = 625
_IN = 128
_HID = 32
_ENC = 64
_NC = 2
_NS = 16
_NW = _NC * _NS
_EPW = _E // _NW
_CHUNK = 128
_NFULL = _EPW // _CHUNK
_TAIL = _EPW - _NFULL * _CHUNK
_RPT = _N // _NS
_GPW = 312
_GCH = 104


def _sc_mesh():
    return plsc.VectorSubcoreMesh(
        core_axis_name="c", subcore_axis_name="s",
        num_cores=_NC, num_subcores=_NS)


_SC_PARAMS = pltpu.CompilerParams(use_tc_tiling_on_sc=False,
                                  needs_layout_passes=False)


def _gather_rows(tables, nid, params):
    n_t = len(tables)
    widths = [t.shape[1] for t in tables]

    @functools.partial(
        pl.kernel,
        out_type=tuple(
            jax.ShapeDtypeStruct((_N, w), jnp.float32) for w in widths),
        mesh=_sc_mesh(),
        scratch_types=(
            [pltpu.VMEM((_GCH,), jnp.int32),
             pltpu.VMEM((16,), jnp.int32)]
            + [pltpu.VMEM((_GCH, w), jnp.float32) for w in widths]
            + [pltpu.VMEM((16, w), jnp.float32) for w in widths]
            + [pltpu.SemaphoreType.DMA]
        ),
        compiler_params=params,
    )
    def k(*refs):
        embs = refs[:n_t]
        nid_h = refs[n_t]
        outs = refs[n_t + 1:2 * n_t + 1]
        idx_v, idx_t = refs[2 * n_t + 1:2 * n_t + 3]
        bufs = refs[2 * n_t + 3:3 * n_t + 3]
        tbufs = refs[3 * n_t + 3:4 * n_t + 3]
        sem = refs[-1]
        wid = lax.axis_index("s") * _NC + lax.axis_index("c")
        base = wid * _GPW
        for ci in range(_GPW // _GCH):
            off = base + ci * _GCH
            pltpu.sync_copy(nid_h.at[pl.ds(off, _GCH)], idx_v)
            cps = [pltpu.async_copy(e.at[idx_v], b, sem)
                   for e, b in zip(embs, bufs)]
            for c in cps:
                c.wait()
            for b, o in zip(bufs, outs):
                pltpu.sync_copy(b, o.at[pl.ds(off, _GCH)])

        @pl.when(wid == 0)
        def _tail():
            off = _NW * _GPW
            pltpu.sync_copy(nid_h.at[pl.ds(off, 16)], idx_t)
            cps = [pltpu.async_copy(e.at[idx_t], b, sem)
                   for e, b in zip(embs, tbufs)]
            for c in cps:
                c.wait()
            for b, o in zip(tbufs, outs):
                pltpu.sync_copy(b, o.at[pl.ds(off, 16)])

    return k(*tables, nid)


def _dense_node(G0, G1, G2, Wf0, Wf1, Wf2, W1a, W1b, Wenc1):
    blk = 1000

    def body(g0, g1, g2, wf0, wf1, wf2, w1a, w1b, we1,
             p0, q0, p1, q1, xw, f2o):
        f0 = jnp.maximum(jnp.dot(g0[...], wf0[...],
                                 preferred_element_type=jnp.float32), 0.0)
        f1 = jnp.maximum(jnp.dot(g1[...], wf1[...],
                                 preferred_element_type=jnp.float32), 0.0)
        f2 = jnp.maximum(jnp.dot(g2[...], wf2[...],
                                 preferred_element_type=jnp.float32), 0.0)
        p0[...] = jnp.dot(f0, w1a[...], preferred_element_type=jnp.float32)
        q0[...] = jnp.dot(f1, w1b[...], preferred_element_type=jnp.float32)
        p1[...] = jnp.dot(f1, w1a[...], preferred_element_type=jnp.float32)
        q1[...] = jnp.dot(f2, w1b[...], preferred_element_type=jnp.float32)
        xw[...] = jnp.dot(g0[...], we1[...],
                          preferred_element_type=jnp.float32)
        f2o[...] = f2

    full = lambda a, b: pl.BlockSpec((a, b), lambda i: (0, 0))
    row = lambda w: pl.BlockSpec((blk, w), lambda i: (i, 0))
    outs = [jax.ShapeDtypeStruct((_N, _HID), jnp.float32)] * 4 + [
        jax.ShapeDtypeStruct((_N, _ENC), jnp.float32),
        jax.ShapeDtypeStruct((_N, _HID), jnp.float32)]
    return pl.pallas_call(
        body,
        grid=(_N // blk,),
        in_specs=[row(128), row(256), row(64),
                  full(128, _HID), full(256, _HID), full(64, _HID),
                  full(_HID, _HID), full(_HID, _HID), full(128, _ENC)],
        out_specs=[row(_HID), row(_HID), row(_HID), row(_HID),
                   row(_ENC), row(_HID)],
        out_shape=outs,
    )(G0, G1, G2, Wf0, Wf1, Wf2, W1a, W1b, Wenc1)


def _rb_tc(F2o, W1c, b1r):
    def body(f, w, b, o):
        o[...] = jnp.dot(f[...], w[...],
                         preferred_element_type=jnp.float32) + b[...]

    return pl.pallas_call(
        body,
        out_shape=jax.ShapeDtypeStruct((_B, _HID), jnp.float32),
    )(F2o, W1c, b1r)


def _dense_mid(HP, W):
    blk = 1000

    def body(hp, w, o):
        h = jnp.maximum(hp[0] + hp[1], 0.0)
        o[...] = jnp.dot(h, w[...], preferred_element_type=jnp.float32)

    return pl.pallas_call(
        body,
        grid=(_N // blk,),
        in_specs=[pl.BlockSpec((2, blk, _ENC), lambda i: (0, i, 0)),
                  pl.BlockSpec((_ENC, _ENC), lambda i: (0, 0))],
        out_specs=pl.BlockSpec((blk, _ENC), lambda i: (i, 0)),
        out_shape=jax.ShapeDtypeStruct((_N, _ENC), jnp.float32),
    )(HP, W)


def _final_tc(H2o, Wout):
    def body(h, w, o):
        o[...] = jnp.dot(h[0] + h[1], w[...],
                         preferred_element_type=jnp.float32)

    return pl.pallas_call(
        body,
        out_shape=jax.ShapeDtypeStruct((_B, 8), jnp.float32),
    )(H2o, Wout)


def _edge_pass(P, Q, V, RB1f, consts, esrc, edst, ehop, zrows, hop_sel,
               sparse_out, flags=None):
    if sparse_out:
        out_type = jax.ShapeDtypeStruct((_NC * _B, _ENC), jnp.float32)
        extra_in = (flags,)
    else:
        out_type = (jax.ShapeDtypeStruct((_NC * _N, _ENC), jnp.float32),
                    jax.ShapeDtypeStruct((_NW * 40,), jnp.int32))
        extra_in = ()

    @functools.partial(
        pl.kernel,
        out_type=out_type,
        mesh=_sc_mesh(),
        scratch_types=[
            [pltpu.VMEM((_CHUNK,), jnp.int32)] * 2,
            [pltpu.VMEM((_CHUNK,), jnp.int32)] * 2,
            [pltpu.VMEM((_CHUNK,), jnp.int32)] * 2,
            pltpu.VMEM((16,), jnp.int32),
            pltpu.VMEM((16,), jnp.int32),
            pltpu.VMEM((16,), jnp.int32),
            [pltpu.VMEM((_CHUNK, _HID), jnp.float32)] * 2,
            [pltpu.VMEM((_CHUNK, _HID), jnp.float32)] * 2,
            [pltpu.VMEM((_CHUNK, _ENC), jnp.float32)] * 2,
            pltpu.VMEM((_CHUNK, _ENC), jnp.float32),
            pltpu.VMEM((16, _HID), jnp.float32),
            pltpu.VMEM((16, _HID), jnp.float32),
            pltpu.VMEM((16, _ENC), jnp.float32),
            pltpu.VMEM((16, _ENC), jnp.float32),
            pltpu.VMEM((512,), jnp.float32),
            pltpu.VMEM((512,), jnp.float32),
            pltpu.VMEM((48,), jnp.float32),
            pltpu.VMEM((40,), jnp.int32),
            pltpu.VMEM_SHARED((_N, _ENC), jnp.float32),
            [pltpu.SemaphoreType.DMA] * 2,
            [pltpu.SemaphoreType.DMA] * 2,
        ],
        compiler_params=_SC_PARAMS,
    )
    def k(P_h, Q_h, V_h, rb_h, c_h, es_h, ed_h, eh_h, z_h, *refs):
        if sparse_out:
            flags_h, out_h = refs[0], refs[1]
        else:
            out_h, flags_o = refs[0], refs[1]
        (sidx, didx, hbuf, sidx16, didx16, hidx16,
         pbuf, qbuf, vbuf, mbuf, pbuf16, qbuf16, vbuf16, mbuf16,
         tt, rbv, cv, fbuf, hacc, isem, gsem) = refs[2:]
        cid = lax.axis_index("c")
        sid = lax.axis_index("s")
        wid = sid * _NC + cid
        if sparse_out:
            pltpu.sync_copy(z_h.at[pl.ds(0, 1)],
                            hacc.at[pl.ds(sid * _NPG, 1)])
        else:
            pltpu.sync_copy(z_h.at[pl.ds(sid * _RPT, _RPT)],
                            hacc.at[pl.ds(sid * _RPT, _RPT)])
        pltpu.sync_copy(rb_h, rbv)
        pltpu.sync_copy(c_h, cv)
        plsc.subcore_barrier()

        lane = lax.iota(jnp.int32, 16)
        b = wid // 2
        rb_lo = rbv[pl.ds(pl.multiple_of(b * 32, 16), 16)]
        rb_hi = rbv[pl.ds(pl.multiple_of(b * 32 + 16, 16), 16)]
        w2_lo = cv[pl.ds(0, 16)]
        w2_hi = cv[pl.ds(16, 16)]
        b2s = cv[pl.ds(32, 16)][0]
        ebase = wid * _EPW

        def iget(ci_next, nb):
            nxt = jnp.minimum(ci_next, _NFULL - 1)
            nbase = pl.multiple_of(ebase + nxt * _CHUNK, 8)
            pltpu.async_copy(es_h.at[pl.ds(nbase, _CHUNK)], sidx[nb],
                             isem[nb])
            pltpu.async_copy(ed_h.at[pl.ds(nbase, _CHUNK)], didx[nb],
                             isem[nb])
            pltpu.async_copy(eh_h.at[pl.ds(nbase, _CHUNK)], hbuf[nb],
                             isem[nb])

        def iwait(nb):
            pltpu.make_async_copy(es_h.at[pl.ds(0, _CHUNK)], sidx[nb],
                                  isem[nb]).wait()
            pltpu.make_async_copy(ed_h.at[pl.ds(0, _CHUNK)], didx[nb],
                                  isem[nb]).wait()
            pltpu.make_async_copy(eh_h.at[pl.ds(0, _CHUNK)], hbuf[nb],
                                  isem[nb]).wait()

        def g_issue(bb):
            pltpu.async_copy(P_h.at[sidx[bb]], pbuf[bb], gsem[bb])
            pltpu.async_copy(Q_h.at[didx[bb]], qbuf[bb], gsem[bb])
            pltpu.async_copy(V_h.at[sidx[bb]], vbuf[bb], gsem[bb])

        def g_wait(bb):
            pltpu.make_async_copy(P_h.at[sidx[bb]], pbuf[bb],
                                  gsem[bb]).wait()
            pltpu.make_async_copy(Q_h.at[didx[bb]], qbuf[bb],
                                  gsem[bb]).wait()
            pltpu.make_async_copy(V_h.at[sidx[bb]], vbuf[bb],
                                  gsem[bb]).wait()

        def is_target(dv):
            f = dv.astype(jnp.float32) * (1.0 / _NPG) + 0.5
            q = f.astype(jnp.int32)
            return (q * _NPG) == dv

        def active(bb):
            m = is_target(didx[bb][pl.ds(0, 16)])
            for jj in range(1, _CHUNK // 16):
                m = m | is_target(didx[bb][pl.ds(jj * 16, 16)])
            return jnp.any(m)

        def group(pb, qb, vb, mb, hv, e0):
            for le in range(16):
                e = e0 + le
                p_lo = pb[e, pl.ds(0, 16)]
                p_hi = pb[e, pl.ds(16, 16)]
                q_lo = qb[e, pl.ds(0, 16)]
                q_hi = qb[e, pl.ds(16, 16)]
                t_lo = jnp.maximum(p_lo + q_lo + rb_lo, 0.0)
                t_hi = jnp.maximum(p_hi + q_hi + rb_hi, 0.0)
                plsc.store_scatter(tt, [lane * 16 + le], t_lo)
                plsc.store_scatter(tt, [lane * 16 + (256 + le)], t_hi)
            att = jnp.full((16,), 0.0, jnp.float32) + b2s
            for kk in range(16):
                att = att + tt[pl.ds(kk * 16, 16)] * w2_lo[kk]
            for kk in range(16):
                att = att + tt[pl.ds((kk + 16) * 16, 16)] * w2_hi[kk]
            sig = 1.0 / (1.0 + jnp.exp(-att))
            gate = jnp.where(hv == hop_sel, sig, 0.0)
            for le in range(16):
                e = e0 + le
                g_s = gate[le]
                for j in range(_ENC // 16):
                    mb[e, pl.ds(j * 16, 16)] = vb[e, pl.ds(j * 16, 16)] * g_s

        def compute_scatter(bb):
            def group_body(g, __):
                e0 = g * 16
                hv = hbuf[bb][pl.ds(e0, 16)]
                group(pbuf[bb], qbuf[bb], vbuf[bb], mbuf, hv, e0)
                return __

            lax.fori_loop(0, _CHUNK // 16, group_body, 0)
            pltpu.sync_copy(mbuf, hacc.at[didx[bb]], add=True)

        if sparse_out:
            pltpu.sync_copy(flags_h.at[pl.ds(wid * 40, 40)], fbuf)

            def chunk_body(ci, _):
                fl = plsc.load_gather(
                    fbuf, [jnp.full((16,), ci, jnp.int32)])

                @pl.when(jnp.any(fl != 0))
                def _a():
                    base = pl.multiple_of(ebase + ci * _CHUNK, 8)
                    pltpu.async_copy(es_h.at[pl.ds(base, _CHUNK)],
                                     sidx[0], isem[0])
                    pltpu.async_copy(ed_h.at[pl.ds(base, _CHUNK)],
                                     didx[0], isem[0])
                    pltpu.async_copy(eh_h.at[pl.ds(base, _CHUNK)],
                                     hbuf[0], isem[0])
                    iwait(0)
                    g_issue(0)
                    g_wait(0)
                    compute_scatter(0)

                return _

            lax.fori_loop(0, _NFULL, chunk_body, 0)
        else:
            def step(ci, bb):
                nb = 1 - bb
                g_wait(bb)
                iwait(nb)
                g_issue(nb)
                avec = jnp.where(jnp.full((16,), active(bb)),
                                 jnp.ones((16,), jnp.int32),
                                 jnp.zeros((16,), jnp.int32))
                plsc.store_scatter(fbuf,
                                   [jnp.full((16,), ci, jnp.int32)],
                                   avec, mask=lane < 1)
                compute_scatter(bb)
                iget(ci + 2, bb)

            iget(0, 0)
            iwait(0)
            g_issue(0)
            iget(1, 1)

            def pair_body(i, _):
                step(2 * i, 0)
                step(2 * i + 1, 1)
                return _

            lax.fori_loop(0, (_NFULL - 1) // 2, pair_body, 0)
            step(_NFULL - 1, 0)
            g_wait(1)
            iwait(0)
            pltpu.sync_copy(fbuf, flags_o.at[pl.ds(wid * 40, 40)])

        tbase = ebase + _NFULL * _CHUNK
        pltpu.sync_copy(es_h.at[pl.ds(tbase, 8)], sidx[0].at[pl.ds(0, 8)])
        pltpu.sync_copy(ed_h.at[pl.ds(tbase, 8)], didx[0].at[pl.ds(0, 8)])
        pltpu.sync_copy(eh_h.at[pl.ds(tbase, 8)], hbuf[0].at[pl.ds(0, 8)])
        pad = lane < 8
        sidx16[...] = jnp.where(pad, sidx[0][pl.ds(0, 16)], 1)
        didx16[...] = jnp.where(pad, didx[0][pl.ds(0, 16)], 1)
        hidx16[...] = jnp.where(pad, hbuf[0][pl.ds(0, 16)], -1)
        cp = pltpu.async_copy(P_h.at[sidx16], pbuf16, gsem[0])
        cq = pltpu.async_copy(Q_h.at[didx16], qbuf16, gsem[0])
        cvv = pltpu.async_copy(V_h.at[sidx16], vbuf16, gsem[0])
        cp.wait()
        cq.wait()
        cvv.wait()
        group(pbuf16, qbuf16, vbuf16, mbuf16, hidx16[...], 0)
        pltpu.sync_copy(mbuf16, hacc.at[didx16], add=True)

        plsc.subcore_barrier()
        if sparse_out:
            pltpu.sync_copy(hacc.at[pl.ds(sid * _NPG, 1)],
                            out_h.at[pl.ds(cid * _B + sid, 1)])
        else:
            out_off = pl.multiple_of(cid * _N + sid * _RPT, 8)
            pltpu.sync_copy(hacc.at[pl.ds(sid * _RPT, _RPT)],
                            out_h.at[pl.ds(out_off, _RPT)])

    return k(P, Q, V, RB1f, consts, esrc, edst, ehop, zrows, *extra_in)


def kernel(all_emb0, all_emb1, all_emb2, Wf0, Wf1, Wf2, W_ext1, b_ext1,
           W_ext2, b_ext2, W_enc1, W_enc2, W_out, node_ids, edge_src,
           edge_dst, e_hop, labels):
    W1a = W_ext1[0:_HID]
    W1b = W_ext1[_HID:2 * _HID]
    W1c = W_ext1[2 * _HID:3 * _HID]
    consts = jnp.concatenate([W_ext2[:, 0], b_ext2,
                              jnp.zeros((15,), jnp.float32)])
    zeros = jnp.zeros((_N, _ENC), jnp.float32)

    G0, G1 = _gather_rows((all_emb0, all_emb1), node_ids, None)
    (G2,) = _gather_rows((all_emb2,), node_ids, _SC_PARAMS)
    P0, Q0, P1, Q1, XW, F2 = _dense_node(
        G0, G1, G2, Wf0, Wf1, Wf2, W1a, W1b, W_enc1)
    F2o = F2.reshape(_B, _NPG, _HID)[:, 0, :]
    RB1 = _rb_tc(F2o, W1c, b_ext1.reshape(1, _HID))
    RB1f = RB1.reshape(-1)

    H1P, FLG = _edge_pass(P0, Q0, XW, RB1f, consts, edge_src, edge_dst,
                          e_hop, zeros, hop_sel=1, sparse_out=False)
    HW1 = _dense_mid(H1P.reshape(_NC, _N, _ENC), W_enc2)
    H2P = _edge_pass(P1, Q1, HW1, RB1f, consts, edge_src, edge_dst, e_hop,
                     zeros, hop_sel=0, sparse_out=True, flags=FLG)
    return _final_tc(H2P.reshape(_NC, _B, _ENC), W_out)


# speedup vs baseline: 11.4183x; 1.0207x over previous
import functools

import jax
import jax.numpy as jnp
from jax import lax
from jax.experimental import pallas as pl
from jax.experimental.pallas import tpu as pltpu
from jax.experimental.pallas import tpu_sc as plsc

_N_TOTAL = 50000
_N = 10000
_E = 160000
_B = 16
_NPG = 625
_IN = 128
_HID = 32
_ENC = 64
_NC = 2
_NS = 16
_NW = _NC * _NS
_EPW = _E // _NW
_CHUNK = 128
_NFULL = _EPW // _CHUNK
_TAIL = _EPW - _NFULL * _CHUNK
_RPT = _N // _NS
_GPW = 312
_GCH = 104


def _sc_mesh():
    return plsc.VectorSubcoreMesh(
        core_axis_name="c", subcore_axis_name="s",
        num_cores=_NC, num_subcores=_NS)


_SC_PARAMS = pltpu.CompilerParams(use_tc_tiling_on_sc=False,
                                  needs_layout_passes=False)


def _gather_rows(tables, nid, params):
    n_t = len(tables)
    widths = [t.shape[1] for t in tables]

    @functools.partial(
        pl.kernel,
        out_type=tuple(
            jax.ShapeDtypeStruct((_N, w), jnp.float32) for w in widths),
        mesh=_sc_mesh(),
        scratch_types=(
            [pltpu.VMEM((_GCH,), jnp.int32),
             pltpu.VMEM((16,), jnp.int32)]
            + [pltpu.VMEM((_GCH, w), jnp.float32) for w in widths]
            + [pltpu.VMEM((16, w), jnp.float32) for w in widths]
            + [pltpu.SemaphoreType.DMA]
        ),
        compiler_params=params,
    )
    def k(*refs):
        embs = refs[:n_t]
        nid_h = refs[n_t]
        outs = refs[n_t + 1:2 * n_t + 1]
        idx_v, idx_t = refs[2 * n_t + 1:2 * n_t + 3]
        bufs = refs[2 * n_t + 3:3 * n_t + 3]
        tbufs = refs[3 * n_t + 3:4 * n_t + 3]
        sem = refs[-1]
        wid = lax.axis_index("s") * _NC + lax.axis_index("c")
        base = wid * _GPW
        for ci in range(_GPW // _GCH):
            off = base + ci * _GCH
            pltpu.sync_copy(nid_h.at[pl.ds(off, _GCH)], idx_v)
            cps = [pltpu.async_copy(e.at[idx_v], b, sem)
                   for e, b in zip(embs, bufs)]
            for c in cps:
                c.wait()
            for b, o in zip(bufs, outs):
                pltpu.sync_copy(b, o.at[pl.ds(off, _GCH)])

        @pl.when(wid == 0)
        def _tail():
            off = _NW * _GPW
            pltpu.sync_copy(nid_h.at[pl.ds(off, 16)], idx_t)
            cps = [pltpu.async_copy(e.at[idx_t], b, sem)
                   for e, b in zip(embs, tbufs)]
            for c in cps:
                c.wait()
            for b, o in zip(tbufs, outs):
                pltpu.sync_copy(b, o.at[pl.ds(off, 16)])

    return k(*tables, nid)


def _dense_node(G0, G1, G2, Wf0, Wf1, Wf2, W1a, W1b, Wenc1):
    blk = 1000

    def body(g0, g1, g2, wf0, wf1, wf2, w1a, w1b, we1,
             p0, q0, p1, q1, xw, f2o):
        f0 = jnp.maximum(jnp.dot(g0[...], wf0[...],
                                 preferred_element_type=jnp.float32), 0.0)
        f1 = jnp.maximum(jnp.dot(g1[...], wf1[...],
                                 preferred_element_type=jnp.float32), 0.0)
        f2 = jnp.maximum(jnp.dot(g2[...], wf2[...],
                                 preferred_element_type=jnp.float32), 0.0)
        p0[...] = jnp.dot(f0, w1a[...], preferred_element_type=jnp.float32)
        q0[...] = jnp.dot(f1, w1b[...], preferred_element_type=jnp.float32)
        p1[...] = jnp.dot(f1, w1a[...], preferred_element_type=jnp.float32)
        q1[...] = jnp.dot(f2, w1b[...], preferred_element_type=jnp.float32)
        xw[...] = jnp.dot(g0[...], we1[...],
                          preferred_element_type=jnp.float32)
        f2o[...] = f2

    full = lambda a, b: pl.BlockSpec((a, b), lambda i: (0, 0))
    row = lambda w: pl.BlockSpec((blk, w), lambda i: (i, 0))
    outs = [jax.ShapeDtypeStruct((_N, _HID), jnp.float32)] * 4 + [
        jax.ShapeDtypeStruct((_N, _ENC), jnp.float32),
        jax.ShapeDtypeStruct((_N, _HID), jnp.float32)]
    return pl.pallas_call(
        body,
        grid=(_N // blk,),
        in_specs=[row(128), row(256), row(64),
                  full(128, _HID), full(256, _HID), full(64, _HID),
                  full(_HID, _HID), full(_HID, _HID), full(128, _ENC)],
        out_specs=[row(_HID), row(_HID), row(_HID), row(_HID),
                   row(_ENC), row(_HID)],
        out_shape=outs,
    )(G0, G1, G2, Wf0, Wf1, Wf2, W1a, W1b, Wenc1)


def _rb_tc(F2o, W1c, b1r):
    def body(f, w, b, o):
        o[...] = jnp.dot(f[...], w[...],
                         preferred_element_type=jnp.float32) + b[...]

    return pl.pallas_call(
        body,
        out_shape=jax.ShapeDtypeStruct((_B, _HID), jnp.float32),
    )(F2o, W1c, b1r)


def _dense_mid(HP, W):
    blk = 1000

    def body(hp, w, o):
        h = jnp.maximum(hp[0] + hp[1], 0.0)
        o[...] = jnp.dot(h, w[...], preferred_element_type=jnp.float32)

    return pl.pallas_call(
        body,
        grid=(_N // blk,),
        in_specs=[pl.BlockSpec((2, blk, _ENC), lambda i: (0, i, 0)),
                  pl.BlockSpec((_ENC, _ENC), lambda i: (0, 0))],
        out_specs=pl.BlockSpec((blk, _ENC), lambda i: (i, 0)),
        out_shape=jax.ShapeDtypeStruct((_N, _ENC), jnp.float32),
    )(HP, W)


def _final_tc(H2o, Wout):
    def body(h, w, o):
        o[...] = jnp.dot(h[0] + h[1], w[...],
                         preferred_element_type=jnp.float32)

    return pl.pallas_call(
        body,
        out_shape=jax.ShapeDtypeStruct((_B, 8), jnp.float32),
    )(H2o, Wout)


def _edge_pass(P, Q, V, RB1f, consts, esrc, edst, ehop, zrows, hop_sel,
               sparse_out, flags=None):
    if sparse_out:
        out_type = jax.ShapeDtypeStruct((_NC * _B, _ENC), jnp.float32)
        extra_in = (flags,)
    else:
        out_type = (jax.ShapeDtypeStruct((_NC * _N, _ENC), jnp.float32),
                    jax.ShapeDtypeStruct((_NW * 40,), jnp.int32))
        extra_in = ()

    @functools.partial(
        pl.kernel,
        out_type=out_type,
        mesh=_sc_mesh(),
        scratch_types=[
            [pltpu.VMEM((_CHUNK,), jnp.int32)] * 2,
            [pltpu.VMEM((_CHUNK,), jnp.int32)] * 2,
            [pltpu.VMEM((_CHUNK,), jnp.int32)] * 2,
            pltpu.VMEM((16,), jnp.int32),
            pltpu.VMEM((16,), jnp.int32),
            pltpu.VMEM((16,), jnp.int32),
            [pltpu.VMEM((_CHUNK, _HID), jnp.float32)] * 2,
            [pltpu.VMEM((_CHUNK, _HID), jnp.float32)] * 2,
            [pltpu.VMEM((_CHUNK, _ENC), jnp.float32)] * 2,
            [pltpu.VMEM((_CHUNK, _ENC), jnp.float32)] * 2,
            [pltpu.VMEM((_CHUNK,), jnp.int32)] * 2,
            pltpu.VMEM((16, _HID), jnp.float32),
            pltpu.VMEM((16, _HID), jnp.float32),
            pltpu.VMEM((16, _ENC), jnp.float32),
            pltpu.VMEM((16, _ENC), jnp.float32),
            pltpu.VMEM((512,), jnp.float32),
            pltpu.VMEM((512,), jnp.float32),
            pltpu.VMEM((48,), jnp.float32),
            pltpu.VMEM((40,), jnp.int32),
            pltpu.VMEM_SHARED((_N, _ENC), jnp.float32),
            [pltpu.SemaphoreType.DMA] * 2,
            [pltpu.SemaphoreType.DMA] * 2,
            [pltpu.SemaphoreType.DMA] * 2,
        ],
        compiler_params=_SC_PARAMS,
    )
    def k(P_h, Q_h, V_h, rb_h, c_h, es_h, ed_h, eh_h, z_h, *refs):
        if sparse_out:
            flags_h, out_h = refs[0], refs[1]
        else:
            out_h, flags_o = refs[0], refs[1]
        (sidx, didx, hbuf, sidx16, didx16, hidx16,
         pbuf, qbuf, vbuf, mbuf, sdidx, pbuf16, qbuf16, vbuf16, mbuf16,
         tt, rbv, cv, fbuf, hacc, isem, gsem, ssem) = refs[2:]
        cid = lax.axis_index("c")
        sid = lax.axis_index("s")
        wid = sid * _NC + cid
        if sparse_out:
            pltpu.sync_copy(z_h.at[pl.ds(0, 1)],
                            hacc.at[pl.ds(sid * _NPG, 1)])
        else:
            pltpu.sync_copy(z_h.at[pl.ds(sid * _RPT, _RPT)],
                            hacc.at[pl.ds(sid * _RPT, _RPT)])
        pltpu.sync_copy(rb_h, rbv)
        pltpu.sync_copy(c_h, cv)
        plsc.subcore_barrier()

        lane = lax.iota(jnp.int32, 16)
        b = wid // 2
        rb_lo = rbv[pl.ds(pl.multiple_of(b * 32, 16), 16)]
        rb_hi = rbv[pl.ds(pl.multiple_of(b * 32 + 16, 16), 16)]
        w2_lo = cv[pl.ds(0, 16)]
        w2_hi = cv[pl.ds(16, 16)]
        b2s = cv[pl.ds(32, 16)][0]
        ebase = wid * _EPW

        def iget(ci_next, nb):
            nxt = jnp.minimum(ci_next, _NFULL - 1)
            nbase = pl.multiple_of(ebase + nxt * _CHUNK, 8)
            pltpu.async_copy(es_h.at[pl.ds(nbase, _CHUNK)], sidx[nb],
                             isem[nb])
            pltpu.async_copy(ed_h.at[pl.ds(nbase, _CHUNK)], didx[nb],
                             isem[nb])
            pltpu.async_copy(eh_h.at[pl.ds(nbase, _CHUNK)], hbuf[nb],
                             isem[nb])

        def iwait(nb):
            pltpu.make_async_copy(es_h.at[pl.ds(0, _CHUNK)], sidx[nb],
                                  isem[nb]).wait()
            pltpu.make_async_copy(ed_h.at[pl.ds(0, _CHUNK)], didx[nb],
                                  isem[nb]).wait()
            pltpu.make_async_copy(eh_h.at[pl.ds(0, _CHUNK)], hbuf[nb],
                                  isem[nb]).wait()

        def g_issue(bb):
            pltpu.async_copy(P_h.at[sidx[bb]], pbuf[bb], gsem[bb])
            pltpu.async_copy(Q_h.at[didx[bb]], qbuf[bb], gsem[bb])
            pltpu.async_copy(V_h.at[sidx[bb]], vbuf[bb], gsem[bb])

        def g_wait(bb):
            pltpu.make_async_copy(P_h.at[sidx[bb]], pbuf[bb],
                                  gsem[bb]).wait()
            pltpu.make_async_copy(Q_h.at[didx[bb]], qbuf[bb],
                                  gsem[bb]).wait()
            pltpu.make_async_copy(V_h.at[sidx[bb]], vbuf[bb],
                                  gsem[bb]).wait()

        def is_target(dv):
            f = dv.astype(jnp.float32) * (1.0 / _NPG) + 0.5
            q = f.astype(jnp.int32)
            return (q * _NPG) == dv

        def active(bb):
            m = is_target(didx[bb][pl.ds(0, 16)])
            for jj in range(1, _CHUNK // 16):
                m = m | is_target(didx[bb][pl.ds(jj * 16, 16)])
            return jnp.any(m)

        def group(pb, qb, vb, mb, hv, e0):
            for le in range(16):
                e = e0 + le
                p_lo = pb[e, pl.ds(0, 16)]
                p_hi = pb[e, pl.ds(16, 16)]
                q_lo = qb[e, pl.ds(0, 16)]
                q_hi = qb[e, pl.ds(16, 16)]
                t_lo = jnp.maximum(p_lo + q_lo + rb_lo, 0.0)
                t_hi = jnp.maximum(p_hi + q_hi + rb_hi, 0.0)
                plsc.store_scatter(tt, [lane * 16 + le], t_lo)
                plsc.store_scatter(tt, [lane * 16 + (256 + le)], t_hi)
            att = jnp.full((16,), 0.0, jnp.float32) + b2s
            for kk in range(16):
                att = att + tt[pl.ds(kk * 16, 16)] * w2_lo[kk]
            for kk in range(16):
                att = att + tt[pl.ds((kk + 16) * 16, 16)] * w2_hi[kk]
            sig = 1.0 / (1.0 + jnp.exp(-att))
            gate = jnp.where(hv == hop_sel, sig, 0.0)
            for le in range(16):
                e = e0 + le
                g_s = gate[le]
                for j in range(_ENC // 16):
                    mb[e, pl.ds(j * 16, 16)] = vb[e, pl.ds(j * 16, 16)] * g_s

        def compute_scatter(bb, first=False, sync=False):
            if not (first or sync):
                pltpu.make_async_copy(mbuf[bb], hacc.at[sdidx[bb]],
                                      ssem[bb]).wait()

            def group_body(g, __):
                e0 = g * 16
                hv = hbuf[bb][pl.ds(e0, 16)]
                group(pbuf[bb], qbuf[bb], vbuf[bb], mbuf[bb], hv, e0)
                return __

            lax.fori_loop(0, _CHUNK // 16, group_body, 0)
            for jj in range(_CHUNK // 16):
                sdidx[bb][pl.ds(jj * 16, 16)] = didx[bb][pl.ds(jj * 16, 16)]
            if sync:
                pltpu.sync_copy(mbuf[bb], hacc.at[sdidx[bb]], add=True)
            else:
                pltpu.async_copy(mbuf[bb], hacc.at[sdidx[bb]], ssem[bb],
                                 add=True)

        if sparse_out:
            pltpu.sync_copy(flags_h.at[pl.ds(wid * 40, 40)], fbuf)

            def chunk_body(ci, _):
                fl = plsc.load_gather(
                    fbuf, [jnp.full((16,), ci, jnp.int32)])

                @pl.when(jnp.any(fl != 0))
                def _a():
                    base = pl.multiple_of(ebase + ci * _CHUNK, 8)
                    pltpu.async_copy(es_h.at[pl.ds(base, _CHUNK)],
                                     sidx[0], isem[0])
                    pltpu.async_copy(ed_h.at[pl.ds(base, _CHUNK)],
                                     didx[0], isem[0])
                    pltpu.async_copy(eh_h.at[pl.ds(base, _CHUNK)],
                                     hbuf[0], isem[0])
                    iwait(0)
                    g_issue(0)
                    g_wait(0)
                    compute_scatter(0, sync=True)

                return _

            lax.fori_loop(0, _NFULL, chunk_body, 0)
        else:
            def step(ci, bb, first=False):
                nb = 1 - bb
                g_wait(bb)
                iwait(nb)
                g_issue(nb)
                avec = jnp.where(jnp.full((16,), active(bb)),
                                 jnp.ones((16,), jnp.int32),
                                 jnp.zeros((16,), jnp.int32))
                plsc.store_scatter(fbuf,
                                   [jnp.full((16,), ci, jnp.int32)],
                                   avec, mask=lane < 1)
                compute_scatter(bb, first=first)
                iget(ci + 2, bb)

            iget(0, 0)
            iwait(0)
            g_issue(0)
            iget(1, 1)
            step(0, 0, first=True)
            step(1, 1, first=True)

            def pair_body(i, _):
                step(2 * i, 0)
                step(2 * i + 1, 1)
                return _

            lax.fori_loop(1, (_NFULL - 1) // 2, pair_body, 0)
            step(_NFULL - 1, 0)
            g_wait(1)
            iwait(0)
            pltpu.make_async_copy(mbuf[0], hacc.at[sdidx[0]],
                                  ssem[0]).wait()
            pltpu.make_async_copy(mbuf[1], hacc.at[sdidx[1]],
                                  ssem[1]).wait()
            pltpu.sync_copy(fbuf, flags_o.at[pl.ds(wid * 40, 40)])

        tbase = ebase + _NFULL * _CHUNK
        pltpu.sync_copy(es_h.at[pl.ds(tbase, 8)], sidx[0].at[pl.ds(0, 8)])
        pltpu.sync_copy(ed_h.at[pl.ds(tbase, 8)], didx[0].at[pl.ds(0, 8)])
        pltpu.sync_copy(eh_h.at[pl.ds(tbase, 8)], hbuf[0].at[pl.ds(0, 8)])
        pad = lane < 8
        sidx16[...] = jnp.where(pad, sidx[0][pl.ds(0, 16)], 1)
        didx16[...] = jnp.where(pad, didx[0][pl.ds(0, 16)], 1)
        hidx16[...] = jnp.where(pad, hbuf[0][pl.ds(0, 16)], -1)
        cp = pltpu.async_copy(P_h.at[sidx16], pbuf16, gsem[0])
        cq = pltpu.async_copy(Q_h.at[didx16], qbuf16, gsem[0])
        cvv = pltpu.async_copy(V_h.at[sidx16], vbuf16, gsem[0])
        cp.wait()
        cq.wait()
        cvv.wait()
        group(pbuf16, qbuf16, vbuf16, mbuf16, hidx16[...], 0)
        pltpu.sync_copy(mbuf16, hacc.at[didx16], add=True)

        plsc.subcore_barrier()
        if sparse_out:
            pltpu.sync_copy(hacc.at[pl.ds(sid * _NPG, 1)],
                            out_h.at[pl.ds(cid * _B + sid, 1)])
        else:
            out_off = pl.multiple_of(cid * _N + sid * _RPT, 8)
            pltpu.sync_copy(hacc.at[pl.ds(sid * _RPT, _RPT)],
                            out_h.at[pl.ds(out_off, _RPT)])

    return k(P, Q, V, RB1f, consts, esrc, edst, ehop, zrows, *extra_in)


def kernel(all_emb0, all_emb1, all_emb2, Wf0, Wf1, Wf2, W_ext1, b_ext1,
           W_ext2, b_ext2, W_enc1, W_enc2, W_out, node_ids, edge_src,
           edge_dst, e_hop, labels):
    W1a = W_ext1[0:_HID]
    W1b = W_ext1[_HID:2 * _HID]
    W1c = W_ext1[2 * _HID:3 * _HID]
    consts = jnp.concatenate([W_ext2[:, 0], b_ext2,
                              jnp.zeros((15,), jnp.float32)])
    zeros = jnp.zeros((_N, _ENC), jnp.float32)

    G0, G1 = _gather_rows((all_emb0, all_emb1), node_ids, None)
    (G2,) = _gather_rows((all_emb2,), node_ids, _SC_PARAMS)
    P0, Q0, P1, Q1, XW, F2 = _dense_node(
        G0, G1, G2, Wf0, Wf1, Wf2, W1a, W1b, W_enc1)
    F2o = F2.reshape(_B, _NPG, _HID)[:, 0, :]
    RB1 = _rb_tc(F2o, W1c, b_ext1.reshape(1, _HID))
    RB1f = RB1.reshape(-1)

    H1P, FLG = _edge_pass(P0, Q0, XW, RB1f, consts, edge_src, edge_dst,
                          e_hop, zeros, hop_sel=1, sparse_out=False)
    HW1 = _dense_mid(H1P.reshape(_NC, _N, _ENC), W_enc2)
    H2P = _edge_pass(P1, Q1, HW1, RB1f, consts, edge_src, edge_dst, e_hop,
                     zeros, hop_sel=0, sparse_out=True, flags=FLG)
    return _final_tc(H2P.reshape(_NC, _B, _ENC), W_out)


# speedup vs baseline: 11.4680x; 1.0043x over previous
import functools

import jax
import jax.numpy as jnp
from jax import lax
from jax.experimental import pallas as pl
from jax.experimental.pallas import tpu as pltpu
from jax.experimental.pallas import tpu_sc as plsc

_N_TOTAL = 50000
_N = 10000
_E = 160000
_B = 16
_NPG = 625
_IN = 128
_HID = 32
_ENC = 64
_NC = 2
_NS = 16
_NW = _NC * _NS
_EPW = _E // _NW
_CHUNK = 128
_NFULL = _EPW // _CHUNK
_TAIL = _EPW - _NFULL * _CHUNK
_RPT = _N // _NS
_GPW = 312
_GCH = 104


def _sc_mesh():
    return plsc.VectorSubcoreMesh(
        core_axis_name="c", subcore_axis_name="s",
        num_cores=_NC, num_subcores=_NS)


_SC_PARAMS = pltpu.CompilerParams(use_tc_tiling_on_sc=False,
                                  needs_layout_passes=False)


def _gather_rows(tables, nid, params):
    n_t = len(tables)
    widths = [t.shape[1] for t in tables]

    @functools.partial(
        pl.kernel,
        out_type=tuple(
            jax.ShapeDtypeStruct((_N, w), jnp.float32) for w in widths),
        mesh=_sc_mesh(),
        scratch_types=(
            [pltpu.VMEM((_GCH,), jnp.int32),
             pltpu.VMEM((16,), jnp.int32)]
            + [pltpu.VMEM((_GCH, w), jnp.float32) for w in widths]
            + [pltpu.VMEM((16, w), jnp.float32) for w in widths]
            + [pltpu.SemaphoreType.DMA]
        ),
        compiler_params=params,
    )
    def k(*refs):
        embs = refs[:n_t]
        nid_h = refs[n_t]
        outs = refs[n_t + 1:2 * n_t + 1]
        idx_v, idx_t = refs[2 * n_t + 1:2 * n_t + 3]
        bufs = refs[2 * n_t + 3:3 * n_t + 3]
        tbufs = refs[3 * n_t + 3:4 * n_t + 3]
        sem = refs[-1]
        wid = lax.axis_index("s") * _NC + lax.axis_index("c")
        base = wid * _GPW
        for ci in range(_GPW // _GCH):
            off = base + ci * _GCH
            pltpu.sync_copy(nid_h.at[pl.ds(off, _GCH)], idx_v)
            cps = [pltpu.async_copy(e.at[idx_v], b, sem)
                   for e, b in zip(embs, bufs)]
            for c in cps:
                c.wait()
            for b, o in zip(bufs, outs):
                pltpu.sync_copy(b, o.at[pl.ds(off, _GCH)])

        @pl.when(wid == 0)
        def _tail():
            off = _NW * _GPW
            pltpu.sync_copy(nid_h.at[pl.ds(off, 16)], idx_t)
            cps = [pltpu.async_copy(e.at[idx_t], b, sem)
                   for e, b in zip(embs, tbufs)]
            for c in cps:
                c.wait()
            for b, o in zip(tbufs, outs):
                pltpu.sync_copy(b, o.at[pl.ds(off, 16)])

    return k(*tables, nid)


def _dense_node(G0, G1, G2, Wf0, Wf1, Wf2, W1a, W1b, Wenc1):
    blk = 1000

    def body(g0, g1, g2, wf0, wf1, wf2, w1a, w1b, we1,
             pv0, q0, p1, q1, f2o):
        f0 = jnp.maximum(jnp.dot(g0[...], wf0[...],
                                 preferred_element_type=jnp.float32), 0.0)
        f1 = jnp.maximum(jnp.dot(g1[...], wf1[...],
                                 preferred_element_type=jnp.float32), 0.0)
        f2 = jnp.maximum(jnp.dot(g2[...], wf2[...],
                                 preferred_element_type=jnp.float32), 0.0)
        p0 = jnp.dot(f0, w1a[...], preferred_element_type=jnp.float32)
        xw = jnp.dot(g0[...], we1[...], preferred_element_type=jnp.float32)
        pv0[...] = jnp.concatenate([p0, xw], axis=1)
        q0[...] = jnp.dot(f1, w1b[...], preferred_element_type=jnp.float32)
        p1[...] = jnp.dot(f1, w1a[...], preferred_element_type=jnp.float32)
        q1[...] = jnp.dot(f2, w1b[...], preferred_element_type=jnp.float32)
        f2o[...] = f2

    full = lambda a, b: pl.BlockSpec((a, b), lambda i: (0, 0))
    row = lambda w: pl.BlockSpec((blk, w), lambda i: (i, 0))
    outs = [jax.ShapeDtypeStruct((_N, _HID + _ENC), jnp.float32)] + [
        jax.ShapeDtypeStruct((_N, _HID), jnp.float32)] * 4
    return pl.pallas_call(
        body,
        grid=(_N // blk,),
        in_specs=[row(128), row(256), row(64),
                  full(128, _HID), full(256, _HID), full(64, _HID),
                  full(_HID, _HID), full(_HID, _HID), full(128, _ENC)],
        out_specs=[row(_HID + _ENC), row(_HID), row(_HID), row(_HID),
                   row(_HID)],
        out_shape=outs,
    )(G0, G1, G2, Wf0, Wf1, Wf2, W1a, W1b, Wenc1)


def _rb_tc(F2o, W1c, b1r):
    def body(f, w, b, o):
        o[...] = jnp.dot(f[...], w[...],
                         preferred_element_type=jnp.float32) + b[...]

    return pl.pallas_call(
        body,
        out_shape=jax.ShapeDtypeStruct((_B, _HID), jnp.float32),
    )(F2o, W1c, b1r)


def _dense_mid(HP, W, P1):
    blk = 1000

    def body(hp, w, p1, o):
        h = jnp.maximum(hp[0] + hp[1], 0.0)
        hw = jnp.dot(h, w[...], preferred_element_type=jnp.float32)
        o[...] = jnp.concatenate([p1[...], hw], axis=1)

    return pl.pallas_call(
        body,
        grid=(_N // blk,),
        in_specs=[pl.BlockSpec((2, blk, _ENC), lambda i: (0, i, 0)),
                  pl.BlockSpec((_ENC, _ENC), lambda i: (0, 0)),
                  pl.BlockSpec((blk, _HID), lambda i: (i, 0))],
        out_specs=pl.BlockSpec((blk, _HID + _ENC), lambda i: (i, 0)),
        out_shape=jax.ShapeDtypeStruct((_N, _HID + _ENC), jnp.float32),
    )(HP, W, P1)


def _final_tc(H2o, Wout):
    def body(h, w, o):
        o[...] = jnp.dot(h[0] + h[1], w[...],
                         preferred_element_type=jnp.float32)

    return pl.pallas_call(
        body,
        out_shape=jax.ShapeDtypeStruct((_B, 8), jnp.float32),
    )(H2o, Wout)


def _edge_pass(PV, Q, RB1f, consts, esrc, edst, ehop, zrows, hop_sel,
               sparse_out, flags=None):
    if sparse_out:
        out_type = jax.ShapeDtypeStruct((_NC * _B, _ENC), jnp.float32)
        extra_in = (flags,)
    else:
        out_type = (jax.ShapeDtypeStruct((_NC * _N, _ENC), jnp.float32),
                    jax.ShapeDtypeStruct((_NW * 40,), jnp.int32))
        extra_in = ()

    @functools.partial(
        pl.kernel,
        out_type=out_type,
        mesh=_sc_mesh(),
        scratch_types=[
            [pltpu.VMEM((_CHUNK,), jnp.int32)] * 2,
            [pltpu.VMEM((_CHUNK,), jnp.int32)] * 2,
            [pltpu.VMEM((_CHUNK,), jnp.int32)] * 2,
            pltpu.VMEM((16,), jnp.int32),
            pltpu.VMEM((16,), jnp.int32),
            pltpu.VMEM((16,), jnp.int32),
            [pltpu.VMEM((_CHUNK, _HID + _ENC), jnp.float32)] * 2,
            [pltpu.VMEM((_CHUNK, _HID), jnp.float32)] * 2,
            [pltpu.VMEM((_CHUNK, _ENC), jnp.float32)] * 2,
            [pltpu.VMEM((_CHUNK,), jnp.int32)] * 2,
            pltpu.VMEM((16, _HID + _ENC), jnp.float32),
            pltpu.VMEM((16, _HID), jnp.float32),
            pltpu.VMEM((16, _ENC), jnp.float32),
            pltpu.VMEM((512,), jnp.float32),
            pltpu.VMEM((512,), jnp.float32),
            pltpu.VMEM((48,), jnp.float32),
            pltpu.VMEM((40,), jnp.int32),
            pltpu.VMEM_SHARED((_N, _ENC), jnp.float32),
            [pltpu.SemaphoreType.DMA] * 2,
            [pltpu.SemaphoreType.DMA] * 2,
            [pltpu.SemaphoreType.DMA] * 2,
        ],
        compiler_params=_SC_PARAMS,
    )
    def k(PV_h, Q_h, rb_h, c_h, es_h, ed_h, eh_h, z_h, *refs):
        if sparse_out:
            flags_h, out_h = refs[0], refs[1]
        else:
            out_h, flags_o = refs[0], refs[1]
        (sidx, didx, hbuf, sidx16, didx16, hidx16,
         pvbuf, qbuf, mbuf, sdidx, pvbuf16, qbuf16, mbuf16,
         tt, rbv, cv, fbuf, hacc, isem, gsem, ssem) = refs[2:]
        cid = lax.axis_index("c")
        sid = lax.axis_index("s")
        wid = sid * _NC + cid
        if sparse_out:
            pltpu.sync_copy(z_h.at[pl.ds(0, 1)],
                            hacc.at[pl.ds(sid * _NPG, 1)])
        else:
            pltpu.sync_copy(z_h.at[pl.ds(sid * _RPT, _RPT)],
                            hacc.at[pl.ds(sid * _RPT, _RPT)])
        pltpu.sync_copy(rb_h, rbv)
        pltpu.sync_copy(c_h, cv)
        plsc.subcore_barrier()

        lane = lax.iota(jnp.int32, 16)
        b = wid // 2
        rb_lo = rbv[pl.ds(pl.multiple_of(b * 32, 16), 16)]
        rb_hi = rbv[pl.ds(pl.multiple_of(b * 32 + 16, 16), 16)]
        w2_lo = cv[pl.ds(0, 16)]
        w2_hi = cv[pl.ds(16, 16)]
        b2s = cv[pl.ds(32, 16)][0]
        ebase = wid * _EPW

        def iget(ci_next, nb):
            nxt = jnp.minimum(ci_next, _NFULL - 1)
            nbase = pl.multiple_of(ebase + nxt * _CHUNK, 8)
            pltpu.async_copy(es_h.at[pl.ds(nbase, _CHUNK)], sidx[nb],
                             isem[nb])
            pltpu.async_copy(ed_h.at[pl.ds(nbase, _CHUNK)], didx[nb],
                             isem[nb])
            pltpu.async_copy(eh_h.at[pl.ds(nbase, _CHUNK)], hbuf[nb],
                             isem[nb])

        def iwait(nb):
            pltpu.make_async_copy(es_h.at[pl.ds(0, _CHUNK)], sidx[nb],
                                  isem[nb]).wait()
            pltpu.make_async_copy(ed_h.at[pl.ds(0, _CHUNK)], didx[nb],
                                  isem[nb]).wait()
            pltpu.make_async_copy(eh_h.at[pl.ds(0, _CHUNK)], hbuf[nb],
                                  isem[nb]).wait()

        def g_issue(bb):
            pltpu.async_copy(PV_h.at[sidx[bb]], pvbuf[bb], gsem[bb])
            pltpu.async_copy(Q_h.at[didx[bb]], qbuf[bb], gsem[bb])

        def g_wait(bb):
            pltpu.make_async_copy(PV_h.at[sidx[bb]], pvbuf[bb],
                                  gsem[bb]).wait()
            pltpu.make_async_copy(Q_h.at[didx[bb]], qbuf[bb],
                                  gsem[bb]).wait()

        def is_target(dv):
            f = dv.astype(jnp.float32) * (1.0 / _NPG) + 0.5
            q = f.astype(jnp.int32)
            return (q * _NPG) == dv

        def active(bb):
            m = is_target(didx[bb][pl.ds(0, 16)])
            for jj in range(1, _CHUNK // 16):
                m = m | is_target(didx[bb][pl.ds(jj * 16, 16)])
            return jnp.any(m)

        def group(pvb, qb, mb, hv, e0):
            for le in range(16):
                e = e0 + le
                p_lo = pvb[e, pl.ds(0, 16)]
                p_hi = pvb[e, pl.ds(16, 16)]
                q_lo = qb[e, pl.ds(0, 16)]
                q_hi = qb[e, pl.ds(16, 16)]
                t_lo = jnp.maximum(p_lo + q_lo + rb_lo, 0.0)
                t_hi = jnp.maximum(p_hi + q_hi + rb_hi, 0.0)
                plsc.store_scatter(tt, [lane * 16 + le], t_lo)
                plsc.store_scatter(tt, [lane * 16 + (256 + le)], t_hi)
            att = jnp.full((16,), 0.0, jnp.float32) + b2s
            for kk in range(16):
                att = att + tt[pl.ds(kk * 16, 16)] * w2_lo[kk]
            for kk in range(16):
                att = att + tt[pl.ds((kk + 16) * 16, 16)] * w2_hi[kk]
            sig = 1.0 / (1.0 + jnp.exp(-att))
            gate = jnp.where(hv == hop_sel, sig, 0.0)
            for le in range(16):
                e = e0 + le
                g_s = gate[le]
                for j in range(_ENC // 16):
                    mb[e, pl.ds(j * 16, 16)] = (
                        pvb[e, pl.ds(_HID + j * 16, 16)] * g_s)

        def compute_scatter(bb, first=False, sync=False):
            if not (first or sync):
                pltpu.make_async_copy(mbuf[bb], hacc.at[sdidx[bb]],
                                      ssem[bb]).wait()

            def group_body(g, __):
                e0 = g * 16
                hv = hbuf[bb][pl.ds(e0, 16)]
                group(pvbuf[bb], qbuf[bb], mbuf[bb], hv, e0)
                return __

            lax.fori_loop(0, _CHUNK // 16, group_body, 0)
            for jj in range(_CHUNK // 16):
                sdidx[bb][pl.ds(jj * 16, 16)] = didx[bb][pl.ds(jj * 16, 16)]
            if sync:
                pltpu.sync_copy(mbuf[bb], hacc.at[sdidx[bb]], add=True)
            else:
                pltpu.async_copy(mbuf[bb], hacc.at[sdidx[bb]], ssem[bb],
                                 add=True)

        if sparse_out:
            pltpu.sync_copy(flags_h.at[pl.ds(wid * 40, 40)], fbuf)

            def chunk_body(ci, _):
                fl = plsc.load_gather(
                    fbuf, [jnp.full((16,), ci, jnp.int32)])

                @pl.when(jnp.any(fl != 0))
                def _a():
                    base = pl.multiple_of(ebase + ci * _CHUNK, 8)
                    pltpu.async_copy(es_h.at[pl.ds(base, _CHUNK)],
                                     sidx[0], isem[0])
                    pltpu.async_copy(ed_h.at[pl.ds(base, _CHUNK)],
                                     didx[0], isem[0])
                    pltpu.async_copy(eh_h.at[pl.ds(base, _CHUNK)],
                                     hbuf[0], isem[0])
                    iwait(0)
                    g_issue(0)
                    g_wait(0)
                    compute_scatter(0, sync=True)

                return _

            lax.fori_loop(0, _NFULL, chunk_body, 0)
        else:
            def step(ci, bb, first=False):
                nb = 1 - bb
                g_wait(bb)
                iwait(nb)
                g_issue(nb)
                avec = jnp.where(jnp.full((16,), active(bb)),
                                 jnp.ones((16,), jnp.int32),
                                 jnp.zeros((16,), jnp.int32))
                plsc.store_scatter(fbuf,
                                   [jnp.full((16,), ci, jnp.int32)],
                                   avec, mask=lane < 1)
                compute_scatter(bb, first=first)
                iget(ci + 2, bb)

            iget(0, 0)
            iwait(0)
            g_issue(0)
            iget(1, 1)
            step(0, 0, first=True)
            step(1, 1, first=True)

            def pair_body(i, _):
                step(2 * i, 0)
                step(2 * i + 1, 1)
                return _

            lax.fori_loop(1, (_NFULL - 1) // 2, pair_body, 0)
            step(_NFULL - 1, 0)
            g_wait(1)
            iwait(0)
            pltpu.make_async_copy(mbuf[0], hacc.at[sdidx[0]],
                                  ssem[0]).wait()
            pltpu.make_async_copy(mbuf[1], hacc.at[sdidx[1]],
                                  ssem[1]).wait()
            pltpu.sync_copy(fbuf, flags_o.at[pl.ds(wid * 40, 40)])

        tbase = ebase + _NFULL * _CHUNK
        pltpu.sync_copy(es_h.at[pl.ds(tbase, 8)], sidx[0].at[pl.ds(0, 8)])
        pltpu.sync_copy(ed_h.at[pl.ds(tbase, 8)], didx[0].at[pl.ds(0, 8)])
        pltpu.sync_copy(eh_h.at[pl.ds(tbase, 8)], hbuf[0].at[pl.ds(0, 8)])
        pad = lane < 8
        sidx16[...] = jnp.where(pad, sidx[0][pl.ds(0, 16)], 1)
        didx16[...] = jnp.where(pad, didx[0][pl.ds(0, 16)], 1)
        hidx16[...] = jnp.where(pad, hbuf[0][pl.ds(0, 16)], -1)
        cp = pltpu.async_copy(PV_h.at[sidx16], pvbuf16, gsem[0])
        cq = pltpu.async_copy(Q_h.at[didx16], qbuf16, gsem[0])
        cp.wait()
        cq.wait()
        group(pvbuf16, qbuf16, mbuf16, hidx16[...], 0)
        pltpu.sync_copy(mbuf16, hacc.at[didx16], add=True)

        plsc.subcore_barrier()
        if sparse_out:
            pltpu.sync_copy(hacc.at[pl.ds(sid * _NPG, 1)],
                            out_h.at[pl.ds(cid * _B + sid, 1)])
        else:
            out_off = pl.multiple_of(cid * _N + sid * _RPT, 8)
            pltpu.sync_copy(hacc.at[pl.ds(sid * _RPT, _RPT)],
                            out_h.at[pl.ds(out_off, _RPT)])

    return k(PV, Q, RB1f, consts, esrc, edst, ehop, zrows, *extra_in)


def kernel(all_emb0, all_emb1, all_emb2, Wf0, Wf1, Wf2, W_ext1, b_ext1,
           W_ext2, b_ext2, W_enc1, W_enc2, W_out, node_ids, edge_src,
           edge_dst, e_hop, labels):
    W1a = W_ext1[0:_HID]
    W1b = W_ext1[_HID:2 * _HID]
    W1c = W_ext1[2 * _HID:3 * _HID]
    consts = jnp.concatenate([W_ext2[:, 0], b_ext2,
                              jnp.zeros((15,), jnp.float32)])
    zeros = jnp.zeros((_N, _ENC), jnp.float32)

    G0, G1 = _gather_rows((all_emb0, all_emb1), node_ids, None)
    (G2,) = _gather_rows((all_emb2,), node_ids, _SC_PARAMS)
    PV0, Q0, P1, Q1, F2 = _dense_node(
        G0, G1, G2, Wf0, Wf1, Wf2, W1a, W1b, W_enc1)
    F2o = F2.reshape(_B, _NPG, _HID)[:, 0, :]
    RB1 = _rb_tc(F2o, W1c, b_ext1.reshape(1, _HID))
    RB1f = RB1.reshape(-1)

    H1P, FLG = _edge_pass(PV0, Q0, RB1f, consts, edge_src, edge_dst,
                          e_hop, zeros, hop_sel=1, sparse_out=False)
    PV1 = _dense_mid(H1P.reshape(_NC, _N, _ENC), W_enc2, P1)
    H2P = _edge_pass(PV1, Q1, RB1f, consts, edge_src, edge_dst, e_hop,
                     zeros, hop_sel=0, sparse_out=True, flags=FLG)
    return _final_tc(H2P.reshape(_NC, _B, _ENC), W_out)
